# Initial kernel scaffold; baseline (speedup 1.0000x reference)
import jax
import jax.numpy as jnp
from jax.experimental import pallas as pl


def kernel(stock_feat, edge_index, edge_attr, params):
    raise NotImplementedError("write your pallas kernel here")


# speedup vs baseline: 1.5786x; 1.5786x over previous
import functools

import jax
import jax.numpy as jnp
from jax import lax
from jax.experimental import pallas as pl
from jax.experimental.pallas import tpu as pltpu
from jax.experimental.pallas import tpu_sc as plsc

N, E, D_IN, D, DE, L = 10000, 160000, 256, 256, 16, 2

BN = 2000
BED = 3200
NSUB = 16
EPT = E // NSUB
BE = 80
NBLK = EPT // BE
DH = D // 2
NQ = 2560
NPASS = 4
NDUMP = 8
TR = NQ // NSUB
ZBR = 32


def _ln_rows(h, g, b):
    mu = jnp.mean(h, axis=1, keepdims=True)
    var = jnp.mean((h - mu) ** 2, axis=1, keepdims=True)
    return (h - mu) * jax.lax.rsqrt(var + 1e-5) * g + b


def _enc_body(sf, w1, b1, w2, b2, out):
    h = jnp.maximum(sf[...] @ w1[...] + b1[...], 0.0)
    out[...] = h @ w2[...] + b2[...]


def _enc(sf, w1, b1, w2, b2):
    return pl.pallas_call(
        _enc_body,
        grid=(N // BN,),
        in_specs=[
            pl.BlockSpec((BN, D_IN), lambda i: (i, 0)),
            pl.BlockSpec((D_IN, D), lambda i: (0, 0)),
            pl.BlockSpec((1, D), lambda i: (0, 0)),
            pl.BlockSpec((D, D), lambda i: (0, 0)),
            pl.BlockSpec((1, D), lambda i: (0, 0)),
        ],
        out_specs=pl.BlockSpec((BN, D), lambda i: (i, 0)),
        out_shape=jax.ShapeDtypeStruct((N, D), jnp.float32),
    )(sf, w1, b1, w2, b2)


def _pre_node_body(x, wsrc, bsrc, wsm, bsm, wa1, m1, mna, mnb, easrc):
    xb = x[...]
    p = xb @ wsrc[...] + bsrc[...]
    mn = xb @ wsm[...] + bsm[...]
    mna[...] = mn[:, :DH]
    mnb[...] = mn[:, DH:]
    a = jnp.sum(jnp.tanh(p) * wa1[...], axis=1) - m1[0, 0]
    easrc[...] = jnp.exp(a).reshape(1, 1, BN)


def _pre_node(x, wsrc, bsrc, wsm, bsm, wa1, m1):
    g = N // BN
    return pl.pallas_call(
        _pre_node_body,
        grid=(g,),
        in_specs=[
            pl.BlockSpec((BN, D), lambda i: (i, 0)),
            pl.BlockSpec((D, D), lambda i: (0, 0)),
            pl.BlockSpec((1, D), lambda i: (0, 0)),
            pl.BlockSpec((D, D), lambda i: (0, 0)),
            pl.BlockSpec((1, D), lambda i: (0, 0)),
            pl.BlockSpec((1, D), lambda i: (0, 0)),
            pl.BlockSpec((1, 1), lambda i: (0, 0), memory_space=pltpu.SMEM),
        ],
        out_specs=[
            pl.BlockSpec((BN, DH), lambda i: (i, 0)),
            pl.BlockSpec((BN, DH), lambda i: (i, 0)),
            pl.BlockSpec((1, 1, BN), lambda i: (i, 0, 0)),
        ],
        out_shape=[
            jax.ShapeDtypeStruct((N, DH), jnp.float32),
            jax.ShapeDtypeStruct((N, DH), jnp.float32),
            jax.ShapeDtypeStruct((g, 1, BN), jnp.float32),
        ],
    )(x, wsrc, bsrc, wsm, bsm, wa1, m1)


def _pre_edge_body(ea, wedge, bedge, wa3, m3, eae):
    s = jnp.tanh(ea[...] @ wedge[...] + bedge[...])
    a = jnp.sum(s * wa3[...], axis=1) - m3[0, 0]
    eae[...] = jnp.exp(a).reshape(1, 1, BED)


def _pre_edge(ea, wedge, bedge, wa3, m3):
    g = E // BED
    return pl.pallas_call(
        _pre_edge_body,
        grid=(g,),
        in_specs=[
            pl.BlockSpec((BED, DE), lambda i: (i, 0)),
            pl.BlockSpec((DE, D), lambda i: (0, 0)),
            pl.BlockSpec((1, D), lambda i: (0, 0)),
            pl.BlockSpec((1, D), lambda i: (0, 0)),
            pl.BlockSpec((1, 1), lambda i: (0, 0), memory_space=pltpu.SMEM),
        ],
        out_specs=pl.BlockSpec((1, 1, BED), lambda i: (i, 0, 0)),
        out_shape=jax.ShapeDtypeStruct((g, 1, BED), jnp.float32),
    )(ea, wedge, bedge, wa3, m3)


def _post_body(x, aga, agb, aw, k2, cvec, woutx, wouta, bout, g1, b1,
               wp, bp, wsr, wo, bo, g2, b2, xo):
    xb = x[...]
    awb = aw[...]
    den = awb[:, DE:DE + 1]
    a16 = awb[:, :DE]
    agg_un = (jnp.concatenate([aga[...], agb[...]], axis=1)
              + a16 @ k2[...] + den * cvec[...])
    agg = (agg_un / jnp.maximum(den, 1e-30)) * (den > 0.0)
    upd = xb @ woutx[...] + agg @ wouta[...] + bout[...]
    h_ss = _ln_rows(xb + upd, g1[...], b1[...])
    h0 = jnp.tanh(h_ss @ wp[...] + bp[...])
    h1 = jnp.tanh(xb @ wp[...] + bp[...])
    sc0 = jnp.sum(h0 * wsr[...], axis=1, keepdims=True)
    sc1 = jnp.sum(h1 * wsr[...], axis=1, keepdims=True)
    mx = jnp.maximum(sc0, sc1)
    e0 = jnp.exp(sc0 - mx)
    e1 = jnp.exp(sc1 - mx)
    out = (e0 * h_ss + (2.0 * e1) * xb) / (e0 + 2.0 * e1)
    out = out @ wo[...] + bo[...]
    xo[...] = _ln_rows(out, g2[...], b2[...])


def _post(x, aga, agb, aw, k2, cvec, woutx, wouta, bout, g1, b1,
          wp, bp, wsr, wo, bo, g2, b2):
    full = lambda r, c: pl.BlockSpec((r, c), lambda i: (0, 0))
    return pl.pallas_call(
        _post_body,
        grid=(N // BN,),
        in_specs=[
            pl.BlockSpec((BN, D), lambda i: (i, 0)),
            pl.BlockSpec((BN, DH), lambda i: (i, 0)),
            pl.BlockSpec((BN, DH), lambda i: (i, 0)),
            pl.BlockSpec((BN, DH), lambda i: (i, 0)),
            full(DE, D), full(1, D), full(D, D), full(D, D), full(1, D),
            full(1, D), full(1, D), full(D, D), full(1, D), full(1, D),
            full(D, D), full(1, D), full(1, D), full(1, D),
        ],
        out_specs=pl.BlockSpec((BN, D), lambda i: (i, 0)),
        out_shape=jax.ShapeDtypeStruct((N, D), jnp.float32),
    )(x, aga, agb, aw, k2, cvec, woutx, wouta, bout, g1, b1,
      wp, bp, wsr, wo, bo, g2, b2)


def _head_body(x, w1, b1, w2r, b2, out):
    h = jnp.maximum(x[...] @ w1[...] + b1[...], 0.0)
    out[...] = (jnp.sum(h * w2r[...], axis=1) + b2[0, 0]).reshape(1, 1, BN)


def _head(x, w1, b1, w2r, b2):
    g = N // BN
    return pl.pallas_call(
        _head_body,
        grid=(g,),
        in_specs=[
            pl.BlockSpec((BN, D), lambda i: (i, 0)),
            pl.BlockSpec((D, D), lambda i: (0, 0)),
            pl.BlockSpec((1, D), lambda i: (0, 0)),
            pl.BlockSpec((1, D), lambda i: (0, 0)),
            pl.BlockSpec((1, 1), lambda i: (0, 0), memory_space=pltpu.SMEM),
        ],
        out_specs=pl.BlockSpec((1, 1, BN), lambda i: (i, 0, 0)),
        out_shape=jax.ShapeDtypeStruct((g, 1, BN), jnp.float32),
    )(x, w1, b1, w2r, b2)


def _sc_body(edata_hbm, easrc_hbm, attr_hbm,
             mna_hbm, mnb_hbm,
             agga_hbm, aggb_hbm, aw_hbm,
             easrc_v, edata_blk,
             si_blk, di_blk, w_v, rows_v, attr_v, awrow_v,
             zbuf, acc_sh, aw_sh, sem):
    c = lax.axis_index("c")
    s = lax.axis_index("s")
    eb = s * EPT

    pltpu.sync_copy(easrc_hbm, easrc_v)

    zeros16 = jnp.zeros((16,), jnp.float32)

    def _zrow(r, _):
        for j in range(DH // 16):
            zbuf[r, pl.ds(j * 16, 16)] = zeros16
        return 0

    lax.fori_loop(0, ZBR, _zrow, 0)

    def _zaw(r, _):
        for j in range(2, DH // 16):
            awrow_v[r, pl.ds(j * 16, 16)] = zeros16
        return 0

    lax.fori_loop(0, BE, _zaw, 0)

    iota16 = lax.iota(jnp.int32, 16)
    onehot0 = jnp.where(iota16 == 0, 1.0, 0.0).astype(jnp.float32)

    def _process(mn_hbm, agg_hbm, do_aw):
        for p in range(NPASS):
            lo = p * NQ
            for k in range(TR // ZBR):
                pltpu.sync_copy(zbuf, acc_sh.at[pl.ds(s * TR + k * ZBR, ZBR)])
                if do_aw:
                    pltpu.sync_copy(zbuf,
                                    aw_sh.at[pl.ds(s * TR + k * ZBR, ZBR)])

            @pl.when(s == NSUB - 1)
            def _():
                pltpu.sync_copy(zbuf.at[pl.ds(0, NDUMP)],
                                acc_sh.at[pl.ds(NQ, NDUMP)])
                if do_aw:
                    pltpu.sync_copy(zbuf.at[pl.ds(0, NDUMP)],
                                    aw_sh.at[pl.ds(NQ, NDUMP)])

            plsc.subcore_barrier()

            def _blk(b, _):
                off = b * BE
                pltpu.sync_copy(edata_hbm.at[pl.ds((eb + off) * 4, BE * 4)],
                                edata_blk)
                for j in range(BE // 16):
                    base = iota16 * 4 + j * 64
                    siv = plsc.load_gather(edata_blk, [base])
                    div = plsc.load_gather(edata_blk, [base + 1])
                    eaev = plsc.bitcast(
                        plsc.load_gather(edata_blk, [base + 2]), jnp.float32)
                    ea = plsc.load_gather(easrc_v, [siv])
                    w_v[pl.ds(j * 16, 16)] = ea * eaev
                    si_blk[pl.ds(j * 16, 16)] = siv
                    inr = (div >= lo) & (div < lo + NQ)
                    di_blk[pl.ds(j * 16, 16)] = jnp.where(
                        inr, div - lo, NQ + (div & (NDUMP - 1)))
                pltpu.async_copy(mn_hbm.at[si_blk], rows_v, sem).wait()
                if do_aw:
                    pltpu.sync_copy(
                        attr_hbm.at[pl.ds((eb + off) * DE, BE * DE)], attr_v)

                def _srow(i, _):
                    wvec = plsc.load_gather(
                        w_v, [jnp.full((16,), i, jnp.int32)])
                    for j in range(DH // 16):
                        rows_v[i, pl.ds(j * 16, 16)] = (
                            rows_v[i, pl.ds(j * 16, 16)] * wvec)
                    if do_aw:
                        awrow_v[i, pl.ds(0, 16)] = (
                            attr_v[pl.ds(i * DE, 16)] * wvec)
                        awrow_v[i, pl.ds(16, 16)] = wvec * onehot0
                    return 0

                lax.fori_loop(0, BE, _srow, 0)
                pltpu.sync_copy(rows_v, acc_sh.at[di_blk], add=True)
                if do_aw:
                    pltpu.sync_copy(awrow_v, aw_sh.at[di_blk], add=True)
                return 0

            lax.fori_loop(0, NBLK, _blk, 0)
            plsc.subcore_barrier()

            if p < NPASS - 1:
                pltpu.sync_copy(acc_sh.at[pl.ds(s * TR, TR)],
                                agg_hbm.at[pl.ds(lo + s * TR, TR)])
                if do_aw:
                    pltpu.sync_copy(aw_sh.at[pl.ds(s * TR, TR)],
                                    aw_hbm.at[pl.ds(lo + s * TR, TR)])
            else:
                nfull = (N - lo) // TR

                @pl.when(s < nfull)
                def _():
                    pltpu.sync_copy(acc_sh.at[pl.ds(s * TR, TR)],
                                    agg_hbm.at[pl.ds(lo + s * TR, TR)])
                    if do_aw:
                        pltpu.sync_copy(aw_sh.at[pl.ds(s * TR, TR)],
                                        aw_hbm.at[pl.ds(lo + s * TR, TR)])

                rem = N - lo - nfull * TR

                @pl.when(s == nfull)
                def _():
                    pltpu.sync_copy(
                        acc_sh.at[pl.ds(nfull * TR, rem)],
                        agg_hbm.at[pl.ds(lo + nfull * TR, rem)])
                    if do_aw:
                        pltpu.sync_copy(
                            aw_sh.at[pl.ds(nfull * TR, rem)],
                            aw_hbm.at[pl.ds(lo + nfull * TR, rem)])

            plsc.subcore_barrier()

    @pl.when(c == 0)
    def _():
        _process(mna_hbm, agga_hbm, True)

    @pl.when(c == 1)
    def _():
        _process(mnb_hbm, aggb_hbm, False)


def _sc_aggregate(edata, easrc, attr_flat, mna, mnb):
    mesh = plsc.VectorSubcoreMesh(core_axis_name="c", subcore_axis_name="s",
                                  num_cores=2, num_subcores=NSUB)
    k = pl.kernel(
        _sc_body,
        out_type=[
            jax.ShapeDtypeStruct((N, DH), jnp.float32),
            jax.ShapeDtypeStruct((N, DH), jnp.float32),
            jax.ShapeDtypeStruct((N, DH), jnp.float32),
        ],
        mesh=mesh,
        compiler_params=pltpu.CompilerParams(needs_layout_passes=False),
        scratch_types=[
            pltpu.VMEM((N,), jnp.float32),
            pltpu.VMEM((BE * 4,), jnp.int32),
            pltpu.VMEM((BE,), jnp.int32),
            pltpu.VMEM((BE,), jnp.int32),
            pltpu.VMEM((BE,), jnp.float32),
            pltpu.VMEM((BE, DH), jnp.float32),
            pltpu.VMEM((BE * DE,), jnp.float32),
            pltpu.VMEM((BE, DH), jnp.float32),
            pltpu.VMEM((ZBR, DH), jnp.float32),
            pltpu.VMEM_SHARED((NQ + NDUMP, DH), jnp.float32),
            pltpu.VMEM_SHARED((NQ + NDUMP, DH), jnp.float32),
            pltpu.SemaphoreType.DMA,
        ],
    )
    return k(edata, easrc, attr_flat, mna, mnb)


def kernel(stock_feat, edge_index, edge_attr, params):
    p = params
    si = edge_index[0]
    di = edge_index[1]

    r = lambda v: v.reshape(1, -1)

    x = _enc(stock_feat, p['enc_W1'], r(p['enc_b1']), p['enc_W2'], r(p['enc_b2']))

    for l in range(L):
        wsrc, bsrc = p[f'l{l}_Wsrc'], p[f'l{l}_bsrc']
        wedge, bedge = p[f'l{l}_Wedge'], p[f'l{l}_bedge']
        wattn = p[f'l{l}_Wattn']
        wmsg, bmsg = p[f'l{l}_Wmsg'], p[f'l{l}_bmsg']
        wa1, wa3 = wattn[:D, 0], wattn[2 * D:, 0]

        wsm = wsrc @ wmsg
        bsm = bsrc @ wmsg
        k2 = wedge @ wmsg
        cvec = bedge @ wmsg + bmsg
        m1 = jnp.sum(jnp.abs(wa1)).reshape(1, 1)
        m3 = jnp.sum(jnp.abs(wa3)).reshape(1, 1)
        wout = p[f'l{l}_Wout']

        mna, mnb, easrc3 = _pre_node(x, wsrc, r(bsrc), wsm, r(bsm), r(wa1), m1)
        eae3 = _pre_edge(edge_attr, wedge, r(bedge), r(wa3), m3)

        edata = jnp.stack(
            [si, di, jax.lax.bitcast_convert_type(eae3.reshape(E), jnp.int32),
             jnp.zeros((E,), jnp.int32)], axis=1).reshape(E * 4)

        agga, aggb, aw = _sc_aggregate(
            edata, easrc3.reshape(N), edge_attr.reshape(E * DE), mna, mnb)

        x = _post(
            x, agga, aggb, aw, k2, r(cvec), wout[:D], wout[D:], r(p[f'l{l}_bout']),
            r(p[f'l{l}_g']), r(p[f'l{l}_b']),
            p['mp_Wp'], r(p['mp_bp']), r(p['mp_Ws'][:, 0]),
            p['mp_Wo'], r(p['mp_bo']), r(p['mp_g']), r(p['mp_b']))

    logits3 = _head(x, p['head_W1'], r(p['head_b1']), r(p['head_W2'][:, 0]),
                    p['head_b2'].reshape(1, 1))
    return logits3.reshape(N)


# speedup vs baseline: 1.6154x; 1.0233x over previous
import functools

import jax
import jax.numpy as jnp
from jax import lax
from jax.experimental import pallas as pl
from jax.experimental.pallas import tpu as pltpu
from jax.experimental.pallas import tpu_sc as plsc

N, E, D_IN, D, DE, L = 10000, 160000, 256, 256, 16, 2

BN = 2000
BED = 3200
NSUB = 16
EPT = E // NSUB
BE = 80
NBLK = EPT // BE
DH = D // 2
NQ = 2560
NPASS = 4
NP = NQ * NPASS
NDUMP = 8
TR = NQ // NSUB
ZBR = 16
EW = 20


def _ln_rows(h, g, b):
    mu = jnp.mean(h, axis=1, keepdims=True)
    var = jnp.mean((h - mu) ** 2, axis=1, keepdims=True)
    return (h - mu) * jax.lax.rsqrt(var + 1e-5) * g + b


def _enc_body(sf, w1, b1, w2, b2, out):
    h = jnp.maximum(sf[...] @ w1[...] + b1[...], 0.0)
    out[...] = h @ w2[...] + b2[...]


def _enc(sf, w1, b1, w2, b2):
    return pl.pallas_call(
        _enc_body,
        grid=(N // BN,),
        in_specs=[
            pl.BlockSpec((BN, D_IN), lambda i: (i, 0)),
            pl.BlockSpec((D_IN, D), lambda i: (0, 0)),
            pl.BlockSpec((1, D), lambda i: (0, 0)),
            pl.BlockSpec((D, D), lambda i: (0, 0)),
            pl.BlockSpec((1, D), lambda i: (0, 0)),
        ],
        out_specs=pl.BlockSpec((BN, D), lambda i: (i, 0)),
        out_shape=jax.ShapeDtypeStruct((N, D), jnp.float32),
    )(sf, w1, b1, w2, b2)


def _pre_node_body(x, wsrc, bsrc, wsm, bsm, wa1, m1, mna, mnb, easrc):
    xb = x[...]
    p = xb @ wsrc[...] + bsrc[...]
    mn = xb @ wsm[...] + bsm[...]
    mna[...] = mn[:, :DH]
    mnb[...] = mn[:, DH:]
    a = jnp.sum(jnp.tanh(p) * wa1[...], axis=1) - m1[0, 0]
    easrc[...] = jnp.exp(a).reshape(1, 1, BN)


def _pre_node(x, wsrc, bsrc, wsm, bsm, wa1, m1):
    g = N // BN
    return pl.pallas_call(
        _pre_node_body,
        grid=(g,),
        in_specs=[
            pl.BlockSpec((BN, D), lambda i: (i, 0)),
            pl.BlockSpec((D, D), lambda i: (0, 0)),
            pl.BlockSpec((1, D), lambda i: (0, 0)),
            pl.BlockSpec((D, D), lambda i: (0, 0)),
            pl.BlockSpec((1, D), lambda i: (0, 0)),
            pl.BlockSpec((1, D), lambda i: (0, 0)),
            pl.BlockSpec((1, 1), lambda i: (0, 0), memory_space=pltpu.SMEM),
        ],
        out_specs=[
            pl.BlockSpec((BN, DH), lambda i: (i, 0)),
            pl.BlockSpec((BN, DH), lambda i: (i, 0)),
            pl.BlockSpec((1, 1, BN), lambda i: (i, 0, 0)),
        ],
        out_shape=[
            jax.ShapeDtypeStruct((N, DH), jnp.float32),
            jax.ShapeDtypeStruct((N, DH), jnp.float32),
            jax.ShapeDtypeStruct((g, 1, BN), jnp.float32),
        ],
    )(x, wsrc, bsrc, wsm, bsm, wa1, m1)


def _pre_edge_body(ea, wedge, bedge, wa3, m3, eae):
    s = jnp.tanh(ea[...] @ wedge[...] + bedge[...])
    a = jnp.sum(s * wa3[...], axis=1) - m3[0, 0]
    eae[...] = jnp.exp(a).reshape(1, 1, BED)


def _pre_edge(ea, wedge, bedge, wa3, m3):
    g = E // BED
    return pl.pallas_call(
        _pre_edge_body,
        grid=(g,),
        in_specs=[
            pl.BlockSpec((BED, DE), lambda i: (i, 0)),
            pl.BlockSpec((DE, D), lambda i: (0, 0)),
            pl.BlockSpec((1, D), lambda i: (0, 0)),
            pl.BlockSpec((1, D), lambda i: (0, 0)),
            pl.BlockSpec((1, 1), lambda i: (0, 0), memory_space=pltpu.SMEM),
        ],
        out_specs=pl.BlockSpec((1, 1, BED), lambda i: (i, 0, 0)),
        out_shape=jax.ShapeDtypeStruct((g, 1, BED), jnp.float32),
    )(ea, wedge, bedge, wa3, m3)


def _post_body(x, aga, agb, aw, k2, cvec, woutx, wouta, bout, g1, b1,
               wp, bp, wsr, wo, bo, g2, b2, xo):
    xb = x[...]
    awb = aw[...]
    den = awb[:, DE:DE + 1]
    a16 = awb[:, :DE]
    agg_un = (jnp.concatenate([aga[...], agb[...]], axis=1)
              + a16 @ k2[...] + den * cvec[...])
    agg = (agg_un / jnp.maximum(den, 1e-30)) * (den > 0.0)
    upd = xb @ woutx[...] + agg @ wouta[...] + bout[...]
    h_ss = _ln_rows(xb + upd, g1[...], b1[...])
    h0 = jnp.tanh(h_ss @ wp[...] + bp[...])
    h1 = jnp.tanh(xb @ wp[...] + bp[...])
    sc0 = jnp.sum(h0 * wsr[...], axis=1, keepdims=True)
    sc1 = jnp.sum(h1 * wsr[...], axis=1, keepdims=True)
    mx = jnp.maximum(sc0, sc1)
    e0 = jnp.exp(sc0 - mx)
    e1 = jnp.exp(sc1 - mx)
    out = (e0 * h_ss + (2.0 * e1) * xb) / (e0 + 2.0 * e1)
    out = out @ wo[...] + bo[...]
    xo[...] = _ln_rows(out, g2[...], b2[...])


def _post(x, aga, agb, aw, k2, cvec, woutx, wouta, bout, g1, b1,
          wp, bp, wsr, wo, bo, g2, b2):
    full = lambda r, c: pl.BlockSpec((r, c), lambda i: (0, 0))
    return pl.pallas_call(
        _post_body,
        grid=(N // BN,),
        in_specs=[
            pl.BlockSpec((BN, D), lambda i: (i, 0)),
            pl.BlockSpec((BN, DH), lambda i: (i, 0)),
            pl.BlockSpec((BN, DH), lambda i: (i, 0)),
            pl.BlockSpec((BN, DH), lambda i: (i, 0)),
            full(DE, D), full(1, D), full(D, D), full(D, D), full(1, D),
            full(1, D), full(1, D), full(D, D), full(1, D), full(1, D),
            full(D, D), full(1, D), full(1, D), full(1, D),
        ],
        out_specs=pl.BlockSpec((BN, D), lambda i: (i, 0)),
        out_shape=jax.ShapeDtypeStruct((N, D), jnp.float32),
    )(x, aga, agb, aw, k2, cvec, woutx, wouta, bout, g1, b1,
      wp, bp, wsr, wo, bo, g2, b2)


def _head_body(x, w1, b1, w2r, b2, out):
    h = jnp.maximum(x[...] @ w1[...] + b1[...], 0.0)
    out[...] = (jnp.sum(h * w2r[...], axis=1) + b2[0, 0]).reshape(1, 1, BN)


def _head(x, w1, b1, w2r, b2):
    g = N // BN
    return pl.pallas_call(
        _head_body,
        grid=(g,),
        in_specs=[
            pl.BlockSpec((BN, D), lambda i: (i, 0)),
            pl.BlockSpec((D, D), lambda i: (0, 0)),
            pl.BlockSpec((1, D), lambda i: (0, 0)),
            pl.BlockSpec((1, D), lambda i: (0, 0)),
            pl.BlockSpec((1, 1), lambda i: (0, 0), memory_space=pltpu.SMEM),
        ],
        out_specs=pl.BlockSpec((1, 1, BN), lambda i: (i, 0, 0)),
        out_shape=jax.ShapeDtypeStruct((g, 1, BN), jnp.float32),
    )(x, w1, b1, w2r, b2)


def _sc_body(edata_hbm, easrc_hbm,
             mna_hbm, mnb_hbm,
             agga_hbm, aggb_hbm, aw_hbm,
             edataA, edataB, siA, siB, diA, diB, wA, wB, easA, easB,
             rowsA, rowsB, awrowA, awrowB, zbuf,
             acc_sh, aw_sh,
             semEA, semEB, semGA, semGB, semXA, semXB,
             semRA, semRB, semAA, semAB):
    c = lax.axis_index("c")
    s = lax.axis_index("s")
    eb = s * EPT

    zeros16 = jnp.zeros((16,), jnp.float32)
    iota16 = lax.iota(jnp.int32, 16)
    onehot0 = jnp.where(iota16 == 0, 1.0, 0.0).astype(jnp.float32)
    dump16 = jnp.full((16,), NQ, jnp.int32)

    def _zrow(r, _):
        for j in range(DH // 16):
            zbuf[r, pl.ds(j * 16, 16)] = zeros16
        return 0

    lax.fori_loop(0, ZBR, _zrow, 0)

    def _zaw(r, _):
        for j in range(2, DH // 16):
            awrowA[r, pl.ds(j * 16, 16)] = zeros16
            awrowB[r, pl.ds(j * 16, 16)] = zeros16
        return 0

    lax.fori_loop(0, BE, _zaw, 0)

    bufs = ((edataA, siA, diA, wA, easA, rowsA, awrowA,
             semEA, semGA, semXA, semRA, semAA),
            (edataB, siB, diB, wB, easB, rowsB, awrowB,
             semEB, semGB, semXB, semRB, semAB))

    def ed_issue(q, blk):
        pltpu.async_copy(edata_hbm.at[pl.ds((eb + blk * BE) * EW, BE * EW)],
                         bufs[q][0], bufs[q][7])

    def ed_wait(q):
        pltpu.make_async_copy(edata_hbm.at[pl.ds(0, BE * EW)],
                              bufs[q][0], bufs[q][7]).wait()

    def scores(q, lo):
        ed, si_b, di_b, w_b = bufs[q][0], bufs[q][1], bufs[q][2], bufs[q][3]
        for j in range(BE // 16):
            base = (iota16 + j * 16) * EW
            siv = plsc.load_gather(ed, [base])
            div = plsc.load_gather(ed, [base + 1])
            eaev = plsc.bitcast(plsc.load_gather(ed, [base + 2]), jnp.float32)
            w_b[pl.ds(j * 16, 16)] = eaev
            si_b[pl.ds(j * 16, 16)] = siv
            inr = (div >= lo) & (div < lo + NQ)
            di_b[pl.ds(j * 16, 16)] = jnp.where(
                inr, div - lo, NQ + (div & (NDUMP - 1)))

    def g_issue(q, mn_hbm):
        pltpu.async_copy(mn_hbm.at[bufs[q][1]], bufs[q][5], bufs[q][8])
        pltpu.async_copy(easrc_hbm.at[bufs[q][1]], bufs[q][4], bufs[q][9])

    def g_wait(q, mn_hbm):
        pltpu.make_async_copy(mn_hbm.at[bufs[q][1]], bufs[q][5],
                              bufs[q][8]).wait()
        pltpu.make_async_copy(easrc_hbm.at[bufs[q][1]], bufs[q][4],
                              bufs[q][9]).wait()

    def proc(q, do_aw):
        ed, si_b, di_b, w_b, eas_b, rows_b, awrow_b = bufs[q][:7]
        for j in range(BE // 16):
            w_b[pl.ds(j * 16, 16)] = (w_b[pl.ds(j * 16, 16)]
                                      * eas_b[pl.ds(j * 16, 16)])

        def _srow(i, _):
            wvec = plsc.load_gather(w_b, [jnp.full((16,), i, jnp.int32)])
            for j in range(DH // 16):
                rows_b[i, pl.ds(j * 16, 16)] = (
                    rows_b[i, pl.ds(j * 16, 16)] * wvec)

            @pl.when(do_aw)
            def _():
                attrv = plsc.bitcast(
                    plsc.load_gather(
                        ed, [jnp.full((16,), i * EW + 4, jnp.int32) + iota16]),
                    jnp.float32)
                awrow_b[i, pl.ds(0, 16)] = attrv * wvec
                awrow_b[i, pl.ds(16, 16)] = wvec * onehot0

            return 0

        lax.fori_loop(0, BE, _srow, 0)
        pltpu.async_copy(rows_b, acc_sh.at[di_b], bufs[q][10], add=True)

        @pl.when(do_aw)
        def _():
            pltpu.async_copy(awrow_b, aw_sh.at[di_b], bufs[q][11], add=True)

    def r_wait(q, do_aw):
        pltpu.make_async_copy(bufs[q][5], acc_sh.at[bufs[q][2]],
                              bufs[q][10]).wait()

        @pl.when(do_aw)
        def _():
            pltpu.make_async_copy(bufs[q][6], aw_sh.at[bufs[q][2]],
                                  bufs[q][11]).wait()

    def _run(mn_hbm, agg_hbm):
        def _pass(p, _):
            lo = p * NQ
            do_aw = (p & 1) == c

            for k in range(TR // ZBR):
                pltpu.sync_copy(zbuf, acc_sh.at[pl.ds(s * TR + k * ZBR, ZBR)])

            @pl.when(do_aw)
            def _():
                for k in range(TR // ZBR):
                    pltpu.sync_copy(zbuf,
                                    aw_sh.at[pl.ds(s * TR + k * ZBR, ZBR)])

            @pl.when(s == NSUB - 1)
            def _():
                pltpu.sync_copy(zbuf.at[pl.ds(0, NDUMP)],
                                acc_sh.at[pl.ds(NQ, NDUMP)])

                @pl.when(do_aw)
                def _():
                    pltpu.sync_copy(zbuf.at[pl.ds(0, NDUMP)],
                                    aw_sh.at[pl.ds(NQ, NDUMP)])

            plsc.subcore_barrier()

            for q in (0, 1):
                for j in range(BE // 16):
                    bufs[q][2][pl.ds(j * 16, 16)] = dump16
            for q in (0, 1):
                pltpu.async_copy(bufs[q][5], acc_sh.at[bufs[q][2]],
                                 bufs[q][10], add=True)

                @pl.when(do_aw)
                def _(q=q):
                    pltpu.async_copy(bufs[q][6], aw_sh.at[bufs[q][2]],
                                     bufs[q][11], add=True)

            ed_issue(0, 0)
            ed_issue(1, 1)

            def _pair(k, _):
                a = 2 * k
                ed_wait(0)
                r_wait(0, do_aw)
                scores(0, lo)
                g_issue(0, mn_hbm)
                ed_wait(1)
                r_wait(1, do_aw)
                scores(1, lo)
                g_issue(1, mn_hbm)
                g_wait(0, mn_hbm)
                proc(0, do_aw)
                ed_issue(0, a + 2)
                g_wait(1, mn_hbm)
                proc(1, do_aw)
                ed_issue(1, a + 3)
                return 0

            lax.fori_loop(0, (NBLK - 1) // 2, _pair, 0)

            ed_wait(0)
            r_wait(0, do_aw)
            scores(0, lo)
            g_issue(0, mn_hbm)
            g_wait(0, mn_hbm)
            proc(0, do_aw)
            ed_wait(1)
            r_wait(0, do_aw)
            r_wait(1, do_aw)

            plsc.subcore_barrier()
            pltpu.sync_copy(acc_sh.at[pl.ds(s * TR, TR)],
                            agg_hbm.at[pl.ds(lo + s * TR, TR)])

            @pl.when(do_aw)
            def _():
                pltpu.sync_copy(aw_sh.at[pl.ds(s * TR, TR)],
                                aw_hbm.at[pl.ds(lo + s * TR, TR)])

            plsc.subcore_barrier()
            return 0

        lax.fori_loop(0, NPASS, _pass, 0)

    @pl.when(c == 0)
    def _():
        _run(mna_hbm, agga_hbm)

    @pl.when(c == 1)
    def _():
        _run(mnb_hbm, aggb_hbm)


def _sc_aggregate(edata, easrc, mna, mnb):
    mesh = plsc.VectorSubcoreMesh(core_axis_name="c", subcore_axis_name="s",
                                  num_cores=2, num_subcores=NSUB)
    k = pl.kernel(
        _sc_body,
        out_type=[
            jax.ShapeDtypeStruct((NP, DH), jnp.float32),
            jax.ShapeDtypeStruct((NP, DH), jnp.float32),
            jax.ShapeDtypeStruct((NP, DH), jnp.float32),
        ],
        mesh=mesh,
        compiler_params=pltpu.CompilerParams(needs_layout_passes=False),
        scratch_types=(
            [pltpu.VMEM((BE * EW,), jnp.int32) for _ in range(2)]
            + [pltpu.VMEM((BE,), jnp.int32) for _ in range(4)]
            + [pltpu.VMEM((BE,), jnp.float32) for _ in range(4)]
            + [pltpu.VMEM((BE, DH), jnp.float32) for _ in range(4)]
            + [pltpu.VMEM((ZBR, DH), jnp.float32)]
            + [pltpu.VMEM_SHARED((NQ + NDUMP, DH), jnp.float32)] * 2
            + [pltpu.SemaphoreType.DMA] * 10
        ),
    )
    return k(edata, easrc, mna, mnb)


def kernel(stock_feat, edge_index, edge_attr, params):
    p = params
    si = edge_index[0]
    di = edge_index[1]

    r = lambda v: v.reshape(1, -1)

    x = _enc(stock_feat, p['enc_W1'], r(p['enc_b1']), p['enc_W2'], r(p['enc_b2']))

    for l in range(L):
        wsrc, bsrc = p[f'l{l}_Wsrc'], p[f'l{l}_bsrc']
        wedge, bedge = p[f'l{l}_Wedge'], p[f'l{l}_bedge']
        wattn = p[f'l{l}_Wattn']
        wmsg, bmsg = p[f'l{l}_Wmsg'], p[f'l{l}_bmsg']
        wa1, wa3 = wattn[:D, 0], wattn[2 * D:, 0]

        wsm = wsrc @ wmsg
        bsm = bsrc @ wmsg
        k2 = wedge @ wmsg
        cvec = bedge @ wmsg + bmsg
        m1 = jnp.sum(jnp.abs(wa1)).reshape(1, 1)
        m3 = jnp.sum(jnp.abs(wa3)).reshape(1, 1)
        wout = p[f'l{l}_Wout']

        mna, mnb, easrc3 = _pre_node(x, wsrc, r(bsrc), wsm, r(bsm), r(wa1), m1)
        eae3 = _pre_edge(edge_attr, wedge, r(bedge), r(wa3), m3)

        ebase = jnp.stack(
            [si, di, jax.lax.bitcast_convert_type(eae3.reshape(E), jnp.int32),
             jnp.zeros((E,), jnp.int32)], axis=1)
        attr_bits = jax.lax.bitcast_convert_type(edge_attr, jnp.int32)
        edata = jnp.concatenate(
            [jnp.concatenate([ebase, attr_bits], axis=1),
             jnp.zeros((BE, EW), jnp.int32)], axis=0).reshape((E + BE) * EW)

        agga_p, aggb_p, aw_p = _sc_aggregate(edata, easrc3.reshape(N), mna, mnb)
        agga, aggb, aw = agga_p[:N], aggb_p[:N], aw_p[:N]

        x = _post(
            x, agga, aggb, aw, k2, r(cvec), wout[:D], wout[D:], r(p[f'l{l}_bout']),
            r(p[f'l{l}_g']), r(p[f'l{l}_b']),
            p['mp_Wp'], r(p['mp_bp']), r(p['mp_Ws'][:, 0]),
            p['mp_Wo'], r(p['mp_bo']), r(p['mp_g']), r(p['mp_b']))

    logits3 = _head(x, p['head_W1'], r(p['head_b1']), r(p['head_W2'][:, 0]),
                    p['head_b2'].reshape(1, 1))
    return logits3.reshape(N)


# speedup vs baseline: 2.7593x; 1.7081x over previous
import functools

import jax
import jax.numpy as jnp
from jax import lax
from jax.experimental import pallas as pl
from jax.experimental.pallas import tpu as pltpu
from jax.experimental.pallas import tpu_sc as plsc

N, E, D_IN, D, DE, L = 10000, 160000, 256, 256, 16, 2

BN = 2000
BED = 3200
NSUB = 16
EPT = E // NSUB
BE = 80
NBLK = EPT // BE
DH = D // 2
NQ = 5120
NPASS = 2
NP = NQ * NPASS
NDUMP = 8
TR = NQ // NSUB
ZBR = 16
EW = 20


def _ln_rows(h, g, b):
    mu = jnp.mean(h, axis=1, keepdims=True)
    var = jnp.mean((h - mu) ** 2, axis=1, keepdims=True)
    return (h - mu) * jax.lax.rsqrt(var + 1e-5) * g + b


def _enc_body(sf, w1, b1, w2, b2, out):
    h = jnp.maximum(sf[...] @ w1[...] + b1[...], 0.0)
    out[...] = h @ w2[...] + b2[...]


def _enc(sf, w1, b1, w2, b2):
    return pl.pallas_call(
        _enc_body,
        grid=(N // BN,),
        in_specs=[
            pl.BlockSpec((BN, D_IN), lambda i: (i, 0)),
            pl.BlockSpec((D_IN, D), lambda i: (0, 0)),
            pl.BlockSpec((1, D), lambda i: (0, 0)),
            pl.BlockSpec((D, D), lambda i: (0, 0)),
            pl.BlockSpec((1, D), lambda i: (0, 0)),
        ],
        out_specs=pl.BlockSpec((BN, D), lambda i: (i, 0)),
        out_shape=jax.ShapeDtypeStruct((N, D), jnp.float32),
    )(sf, w1, b1, w2, b2)


def _pre_node_body(x, wsrc, bsrc, wsm, bsm, wa1, m1, mna, mnb, easrc):
    xb = x[...]
    p = xb @ wsrc[...] + bsrc[...]
    mn = xb @ wsm[...] + bsm[...]
    mna[...] = mn[:, :DH]
    mnb[...] = mn[:, DH:]
    a = jnp.sum(jnp.tanh(p) * wa1[...], axis=1) - m1[0, 0]
    easrc[...] = jnp.exp(a).reshape(1, 1, BN)


def _pre_node(x, wsrc, bsrc, wsm, bsm, wa1, m1):
    g = N // BN
    return pl.pallas_call(
        _pre_node_body,
        grid=(g,),
        in_specs=[
            pl.BlockSpec((BN, D), lambda i: (i, 0)),
            pl.BlockSpec((D, D), lambda i: (0, 0)),
            pl.BlockSpec((1, D), lambda i: (0, 0)),
            pl.BlockSpec((D, D), lambda i: (0, 0)),
            pl.BlockSpec((1, D), lambda i: (0, 0)),
            pl.BlockSpec((1, D), lambda i: (0, 0)),
            pl.BlockSpec((1, 1), lambda i: (0, 0), memory_space=pltpu.SMEM),
        ],
        out_specs=[
            pl.BlockSpec((BN, DH), lambda i: (i, 0)),
            pl.BlockSpec((BN, DH), lambda i: (i, 0)),
            pl.BlockSpec((1, 1, BN), lambda i: (i, 0, 0)),
        ],
        out_shape=[
            jax.ShapeDtypeStruct((N, DH), jnp.float32),
            jax.ShapeDtypeStruct((N, DH), jnp.float32),
            jax.ShapeDtypeStruct((g, 1, BN), jnp.float32),
        ],
    )(x, wsrc, bsrc, wsm, bsm, wa1, m1)


def _pre_edge_body(ea, wedge, bedge, wa3, m3, eae):
    s = jnp.tanh(ea[...] @ wedge[...] + bedge[...])
    a = jnp.sum(s * wa3[...], axis=1) - m3[0, 0]
    eae[...] = jnp.exp(a).reshape(1, 1, BED)


def _pre_edge(ea, wedge, bedge, wa3, m3):
    g = E // BED
    return pl.pallas_call(
        _pre_edge_body,
        grid=(g,),
        in_specs=[
            pl.BlockSpec((BED, DE), lambda i: (i, 0)),
            pl.BlockSpec((DE, D), lambda i: (0, 0)),
            pl.BlockSpec((1, D), lambda i: (0, 0)),
            pl.BlockSpec((1, D), lambda i: (0, 0)),
            pl.BlockSpec((1, 1), lambda i: (0, 0), memory_space=pltpu.SMEM),
        ],
        out_specs=pl.BlockSpec((1, 1, BED), lambda i: (i, 0, 0)),
        out_shape=jax.ShapeDtypeStruct((g, 1, BED), jnp.float32),
    )(ea, wedge, bedge, wa3, m3)


def _post_body(x, aga, agb, aw, k2, cvec, woutx, wouta, bout, g1, b1,
               wp, bp, wsr, wo, bo, g2, b2, xo):
    xb = x[...]
    awb = aw[...]
    den = awb[:, DE:DE + 1]
    a16 = awb[:, :DE]
    agg_un = (jnp.concatenate([aga[...], agb[...]], axis=1)
              + a16 @ k2[...] + den * cvec[...])
    agg = (agg_un / jnp.maximum(den, 1e-30)) * (den > 0.0)
    upd = xb @ woutx[...] + agg @ wouta[...] + bout[...]
    h_ss = _ln_rows(xb + upd, g1[...], b1[...])
    h0 = jnp.tanh(h_ss @ wp[...] + bp[...])
    h1 = jnp.tanh(xb @ wp[...] + bp[...])
    sc0 = jnp.sum(h0 * wsr[...], axis=1, keepdims=True)
    sc1 = jnp.sum(h1 * wsr[...], axis=1, keepdims=True)
    mx = jnp.maximum(sc0, sc1)
    e0 = jnp.exp(sc0 - mx)
    e1 = jnp.exp(sc1 - mx)
    out = (e0 * h_ss + (2.0 * e1) * xb) / (e0 + 2.0 * e1)
    out = out @ wo[...] + bo[...]
    xo[...] = _ln_rows(out, g2[...], b2[...])


def _post(x, aga, agb, aw, k2, cvec, woutx, wouta, bout, g1, b1,
          wp, bp, wsr, wo, bo, g2, b2):
    full = lambda r, c: pl.BlockSpec((r, c), lambda i: (0, 0))
    return pl.pallas_call(
        _post_body,
        grid=(N // BN,),
        in_specs=[
            pl.BlockSpec((BN, D), lambda i: (i, 0)),
            pl.BlockSpec((BN, DH), lambda i: (i, 0)),
            pl.BlockSpec((BN, DH), lambda i: (i, 0)),
            pl.BlockSpec((BN, DH), lambda i: (i, 0)),
            full(DE, D), full(1, D), full(D, D), full(D, D), full(1, D),
            full(1, D), full(1, D), full(D, D), full(1, D), full(1, D),
            full(D, D), full(1, D), full(1, D), full(1, D),
        ],
        out_specs=pl.BlockSpec((BN, D), lambda i: (i, 0)),
        out_shape=jax.ShapeDtypeStruct((N, D), jnp.float32),
    )(x, aga, agb, aw, k2, cvec, woutx, wouta, bout, g1, b1,
      wp, bp, wsr, wo, bo, g2, b2)


def _head_body(x, w1, b1, w2r, b2, out):
    h = jnp.maximum(x[...] @ w1[...] + b1[...], 0.0)
    out[...] = (jnp.sum(h * w2r[...], axis=1) + b2[0, 0]).reshape(1, 1, BN)


def _head(x, w1, b1, w2r, b2):
    g = N // BN
    return pl.pallas_call(
        _head_body,
        grid=(g,),
        in_specs=[
            pl.BlockSpec((BN, D), lambda i: (i, 0)),
            pl.BlockSpec((D, D), lambda i: (0, 0)),
            pl.BlockSpec((1, D), lambda i: (0, 0)),
            pl.BlockSpec((1, D), lambda i: (0, 0)),
            pl.BlockSpec((1, 1), lambda i: (0, 0), memory_space=pltpu.SMEM),
        ],
        out_specs=pl.BlockSpec((1, 1, BN), lambda i: (i, 0, 0)),
        out_shape=jax.ShapeDtypeStruct((g, 1, BN), jnp.float32),
    )(x, w1, b1, w2r, b2)


def _sc_body(edata_hbm, easrc_hbm,
             mna_hbm, mnb_hbm,
             agga_hbm, aggb_hbm, aw_hbm,
             edataA, edataB, siA, siB, diA, diB, wA, wB, easA, easB,
             rowsA, rowsB, awrowA, awrowB, zbuf,
             acc_sh, aw_sh,
             semEA, semEB, semGA, semGB, semXA, semXB,
             semRA, semRB, semAA, semAB):
    c = lax.axis_index("c")
    s = lax.axis_index("s")
    eb = s * EPT

    zeros16 = jnp.zeros((16,), jnp.float32)
    iota16 = lax.iota(jnp.int32, 16)
    onehot0 = jnp.where(iota16 == 0, 1.0, 0.0).astype(jnp.float32)
    dump16 = jnp.full((16,), NQ, jnp.int32)

    def _zrow(r, _):
        for j in range(DH // 16):
            zbuf[r, pl.ds(j * 16, 16)] = zeros16
        return 0

    lax.fori_loop(0, ZBR, _zrow, 0)

    def _zaw(r, _):
        for j in range(2, DH // 16):
            awrowA[r, pl.ds(j * 16, 16)] = zeros16
            awrowB[r, pl.ds(j * 16, 16)] = zeros16
        return 0

    lax.fori_loop(0, BE, _zaw, 0)

    bufs = ((edataA, siA, diA, wA, easA, rowsA, awrowA,
             semEA, semGA, semXA, semRA, semAA),
            (edataB, siB, diB, wB, easB, rowsB, awrowB,
             semEB, semGB, semXB, semRB, semAB))

    def ed_issue(q, blk):
        pltpu.async_copy(edata_hbm.at[pl.ds((eb + blk * BE) * EW, BE * EW)],
                         bufs[q][0], bufs[q][7])

    def ed_wait(q):
        pltpu.make_async_copy(edata_hbm.at[pl.ds(0, BE * EW)],
                              bufs[q][0], bufs[q][7]).wait()

    def scores(q, lo):
        ed, si_b, di_b, w_b = bufs[q][0], bufs[q][1], bufs[q][2], bufs[q][3]
        for j in range(BE // 16):
            base = (iota16 + j * 16) * EW
            siv = plsc.load_gather(ed, [base])
            div = plsc.load_gather(ed, [base + 1])
            eaev = plsc.bitcast(plsc.load_gather(ed, [base + 2]), jnp.float32)
            w_b[pl.ds(j * 16, 16)] = eaev
            si_b[pl.ds(j * 16, 16)] = siv
            inr = (div >= lo) & (div < lo + NQ)
            di_b[pl.ds(j * 16, 16)] = jnp.where(
                inr, div - lo, NQ + (div & (NDUMP - 1)))

    def g_issue(q, mn_hbm):
        pltpu.async_copy(mn_hbm.at[bufs[q][1]], bufs[q][5], bufs[q][8])
        pltpu.async_copy(easrc_hbm.at[bufs[q][1]], bufs[q][4], bufs[q][9])

    def g_wait(q, mn_hbm):
        pltpu.make_async_copy(mn_hbm.at[bufs[q][1]], bufs[q][5],
                              bufs[q][8]).wait()
        pltpu.make_async_copy(easrc_hbm.at[bufs[q][1]], bufs[q][4],
                              bufs[q][9]).wait()

    def proc(q, do_aw):
        ed, si_b, di_b, w_b, eas_b, rows_b, awrow_b = bufs[q][:7]
        for j in range(BE // 16):
            w_b[pl.ds(j * 16, 16)] = (w_b[pl.ds(j * 16, 16)]
                                      * eas_b[pl.ds(j * 16, 16)])

        def _srow(i, _):
            wvec = plsc.load_gather(w_b, [jnp.full((16,), i, jnp.int32)])
            for j in range(DH // 16):
                rows_b[i, pl.ds(j * 16, 16)] = (
                    rows_b[i, pl.ds(j * 16, 16)] * wvec)

            @pl.when(do_aw)
            def _():
                attrv = plsc.bitcast(
                    plsc.load_gather(
                        ed, [jnp.full((16,), i * EW + 4, jnp.int32) + iota16]),
                    jnp.float32)
                awrow_b[i, pl.ds(0, 16)] = attrv * wvec
                awrow_b[i, pl.ds(16, 16)] = wvec * onehot0

            return 0

        lax.fori_loop(0, BE, _srow, 0)
        pltpu.async_copy(rows_b, acc_sh.at[di_b], bufs[q][10], add=True)

        @pl.when(do_aw)
        def _():
            pltpu.async_copy(awrow_b, aw_sh.at[di_b], bufs[q][11], add=True)

    def r_wait(q, do_aw):
        pltpu.make_async_copy(bufs[q][5], acc_sh.at[bufs[q][2]],
                              bufs[q][10]).wait()

        @pl.when(do_aw)
        def _():
            pltpu.make_async_copy(bufs[q][6], aw_sh.at[bufs[q][2]],
                                  bufs[q][11]).wait()

    def _run(mn_hbm, agg_hbm):
        def _pass(p, _):
            lo = p * NQ
            do_aw = (p & 1) == c

            for k in range(TR // ZBR):
                pltpu.sync_copy(zbuf, acc_sh.at[pl.ds(s * TR + k * ZBR, ZBR)])

            @pl.when(do_aw)
            def _():
                for k in range(TR // ZBR):
                    pltpu.sync_copy(zbuf,
                                    aw_sh.at[pl.ds(s * TR + k * ZBR, ZBR)])

            @pl.when(s == NSUB - 1)
            def _():
                pltpu.sync_copy(zbuf.at[pl.ds(0, NDUMP)],
                                acc_sh.at[pl.ds(NQ, NDUMP)])

                @pl.when(do_aw)
                def _():
                    pltpu.sync_copy(zbuf.at[pl.ds(0, NDUMP)],
                                    aw_sh.at[pl.ds(NQ, NDUMP)])

            plsc.subcore_barrier()

            for q in (0, 1):
                for j in range(BE // 16):
                    bufs[q][2][pl.ds(j * 16, 16)] = dump16
            for q in (0, 1):
                pltpu.async_copy(bufs[q][5], acc_sh.at[bufs[q][2]],
                                 bufs[q][10], add=True)

                @pl.when(do_aw)
                def _(q=q):
                    pltpu.async_copy(bufs[q][6], aw_sh.at[bufs[q][2]],
                                     bufs[q][11], add=True)

            ed_issue(0, 0)
            ed_issue(1, 1)

            def _pair(k, _):
                a = 2 * k
                ed_wait(0)
                r_wait(0, do_aw)
                scores(0, lo)
                g_issue(0, mn_hbm)
                ed_wait(1)
                r_wait(1, do_aw)
                scores(1, lo)
                g_issue(1, mn_hbm)
                g_wait(0, mn_hbm)
                proc(0, do_aw)
                ed_issue(0, a + 2)
                g_wait(1, mn_hbm)
                proc(1, do_aw)
                ed_issue(1, a + 3)
                return 0

            lax.fori_loop(0, (NBLK - 1) // 2, _pair, 0)

            ed_wait(0)
            r_wait(0, do_aw)
            scores(0, lo)
            g_issue(0, mn_hbm)
            g_wait(0, mn_hbm)
            proc(0, do_aw)
            ed_wait(1)
            r_wait(0, do_aw)
            r_wait(1, do_aw)

            plsc.subcore_barrier()
            pltpu.sync_copy(acc_sh.at[pl.ds(s * TR, TR)],
                            agg_hbm.at[pl.ds(lo + s * TR, TR)])

            @pl.when(do_aw)
            def _():
                pltpu.sync_copy(aw_sh.at[pl.ds(s * TR, TR)],
                                aw_hbm.at[pl.ds(lo + s * TR, TR)])

            plsc.subcore_barrier()
            return 0

        lax.fori_loop(0, NPASS, _pass, 0)

    @pl.when(c == 0)
    def _():
        _run(mna_hbm, agga_hbm)

    @pl.when(c == 1)
    def _():
        _run(mnb_hbm, aggb_hbm)


def _sc_aggregate(edata, easrc, mna, mnb):
    mesh = plsc.VectorSubcoreMesh(core_axis_name="c", subcore_axis_name="s",
                                  num_cores=2, num_subcores=NSUB)
    k = pl.kernel(
        _sc_body,
        out_type=[
            jax.ShapeDtypeStruct((NP, DH), jnp.float32),
            jax.ShapeDtypeStruct((NP, DH), jnp.float32),
            jax.ShapeDtypeStruct((NP, DH), jnp.float32),
        ],
        mesh=mesh,
        compiler_params=pltpu.CompilerParams(needs_layout_passes=False),
        scratch_types=(
            [pltpu.VMEM((BE * EW,), jnp.int32) for _ in range(2)]
            + [pltpu.VMEM((BE,), jnp.int32) for _ in range(4)]
            + [pltpu.VMEM((BE,), jnp.float32) for _ in range(4)]
            + [pltpu.VMEM((BE, DH), jnp.float32) for _ in range(4)]
            + [pltpu.VMEM((ZBR, DH), jnp.float32)]
            + [pltpu.VMEM_SHARED((NQ + NDUMP, DH), jnp.float32)] * 2
            + [pltpu.SemaphoreType.DMA] * 10
        ),
    )
    return k(edata, easrc, mna, mnb)


def kernel(stock_feat, edge_index, edge_attr, params):
    p = params
    si = edge_index[0]
    di = edge_index[1]

    r = lambda v: v.reshape(1, -1)

    x = _enc(stock_feat, p['enc_W1'], r(p['enc_b1']), p['enc_W2'], r(p['enc_b2']))

    for l in range(L):
        wsrc, bsrc = p[f'l{l}_Wsrc'], p[f'l{l}_bsrc']
        wedge, bedge = p[f'l{l}_Wedge'], p[f'l{l}_bedge']
        wattn = p[f'l{l}_Wattn']
        wmsg, bmsg = p[f'l{l}_Wmsg'], p[f'l{l}_bmsg']
        wa1, wa3 = wattn[:D, 0], wattn[2 * D:, 0]

        wsm = wsrc @ wmsg
        bsm = bsrc @ wmsg
        k2 = wedge @ wmsg
        cvec = bedge @ wmsg + bmsg
        m1 = jnp.sum(jnp.abs(wa1)).reshape(1, 1)
        m3 = jnp.sum(jnp.abs(wa3)).reshape(1, 1)
        wout = p[f'l{l}_Wout']

        mna, mnb, easrc3 = _pre_node(x, wsrc, r(bsrc), wsm, r(bsm), r(wa1), m1)
        eae3 = _pre_edge(edge_attr, wedge, r(bedge), r(wa3), m3)

        ebase = jnp.stack(
            [si, di, jax.lax.bitcast_convert_type(eae3.reshape(E), jnp.int32),
             jnp.zeros((E,), jnp.int32)], axis=1)
        attr_bits = jax.lax.bitcast_convert_type(edge_attr, jnp.int32)
        edata = jnp.concatenate(
            [jnp.concatenate([ebase, attr_bits], axis=1),
             jnp.zeros((BE, EW), jnp.int32)], axis=0).reshape((E + BE) * EW)

        agga_p, aggb_p, aw_p = _sc_aggregate(edata, easrc3.reshape(N), mna, mnb)
        agga, aggb, aw = agga_p[:N], aggb_p[:N], aw_p[:N]

        x = _post(
            x, agga, aggb, aw, k2, r(cvec), wout[:D], wout[D:], r(p[f'l{l}_bout']),
            r(p[f'l{l}_g']), r(p[f'l{l}_b']),
            p['mp_Wp'], r(p['mp_bp']), r(p['mp_Ws'][:, 0]),
            p['mp_Wo'], r(p['mp_bo']), r(p['mp_g']), r(p['mp_b']))

    logits3 = _head(x, p['head_W1'], r(p['head_b1']), r(p['head_W2'][:, 0]),
                    p['head_b2'].reshape(1, 1))
    return logits3.reshape(N)


# speedup vs baseline: 3.1511x; 1.1420x over previous
import functools

import jax
import jax.numpy as jnp
from jax import lax
from jax.experimental import pallas as pl
from jax.experimental.pallas import tpu as pltpu
from jax.experimental.pallas import tpu_sc as plsc

N, E, D_IN, D, DE, L = 10000, 160000, 256, 256, 16, 2

BN = 2000
BED = 3200
NSUB = 16
EPT = E // NSUB
BE = 80
NBLK = EPT // BE
DH = D // 2
NQ = 5120
NPASS = 2
NP = NQ * NPASS
NDUMP = 8
TR = NQ // NSUB
ZBR = 16
EW = 4
PEND = 192


def _ln_rows(h, g, b):
    mu = jnp.mean(h, axis=1, keepdims=True)
    var = jnp.mean((h - mu) ** 2, axis=1, keepdims=True)
    return (h - mu) * jax.lax.rsqrt(var + 1e-5) * g + b


def _enc_body(sf, w1, b1, w2, b2, out):
    h = jnp.maximum(sf[...] @ w1[...] + b1[...], 0.0)
    out[...] = h @ w2[...] + b2[...]


def _enc(sf, w1, b1, w2, b2):
    return pl.pallas_call(
        _enc_body,
        grid=(N // BN,),
        in_specs=[
            pl.BlockSpec((BN, D_IN), lambda i: (i, 0)),
            pl.BlockSpec((D_IN, D), lambda i: (0, 0)),
            pl.BlockSpec((1, D), lambda i: (0, 0)),
            pl.BlockSpec((D, D), lambda i: (0, 0)),
            pl.BlockSpec((1, D), lambda i: (0, 0)),
        ],
        out_specs=pl.BlockSpec((BN, D), lambda i: (i, 0)),
        out_shape=jax.ShapeDtypeStruct((N, D), jnp.float32),
    )(sf, w1, b1, w2, b2)


def _pre_node_body(x, wsrc, bsrc, wsm, bsm, wa1, m1, mna, mnb, easrc):
    xb = x[...]
    p = xb @ wsrc[...] + bsrc[...]
    mn = xb @ wsm[...] + bsm[...]
    mna[...] = mn[:, :DH]
    mnb[...] = mn[:, DH:]
    a = jnp.sum(jnp.tanh(p) * wa1[...], axis=1) - m1[0, 0]
    easrc[...] = jnp.exp(a).reshape(1, 1, BN)


def _pre_node(x, wsrc, bsrc, wsm, bsm, wa1, m1):
    g = N // BN
    return pl.pallas_call(
        _pre_node_body,
        grid=(g,),
        in_specs=[
            pl.BlockSpec((BN, D), lambda i: (i, 0)),
            pl.BlockSpec((D, D), lambda i: (0, 0)),
            pl.BlockSpec((1, D), lambda i: (0, 0)),
            pl.BlockSpec((D, D), lambda i: (0, 0)),
            pl.BlockSpec((1, D), lambda i: (0, 0)),
            pl.BlockSpec((1, D), lambda i: (0, 0)),
            pl.BlockSpec((1, 1), lambda i: (0, 0), memory_space=pltpu.SMEM),
        ],
        out_specs=[
            pl.BlockSpec((BN, DH), lambda i: (i, 0)),
            pl.BlockSpec((BN, DH), lambda i: (i, 0)),
            pl.BlockSpec((1, 1, BN), lambda i: (i, 0, 0)),
        ],
        out_shape=[
            jax.ShapeDtypeStruct((N, DH), jnp.float32),
            jax.ShapeDtypeStruct((N, DH), jnp.float32),
            jax.ShapeDtypeStruct((g, 1, BN), jnp.float32),
        ],
    )(x, wsrc, bsrc, wsm, bsm, wa1, m1)


def _pre_edge_body(ea, wedge, bedge, wa3, m3, eae, awsrc):
    eab = ea[...]
    s = jnp.tanh(eab @ wedge[...] + bedge[...])
    a = jnp.sum(s * wa3[...], axis=1) - m3[0, 0]
    eae[...] = jnp.exp(a).reshape(1, 1, BED)
    awsrc[...] = jnp.concatenate(
        [eab, jnp.ones((BED, 1), jnp.float32),
         jnp.zeros((BED, DH - DE - 1), jnp.float32)], axis=1)


def _pre_edge(ea, wedge, bedge, wa3, m3):
    g = E // BED
    return pl.pallas_call(
        _pre_edge_body,
        grid=(g,),
        in_specs=[
            pl.BlockSpec((BED, DE), lambda i: (i, 0)),
            pl.BlockSpec((DE, D), lambda i: (0, 0)),
            pl.BlockSpec((1, D), lambda i: (0, 0)),
            pl.BlockSpec((1, D), lambda i: (0, 0)),
            pl.BlockSpec((1, 1), lambda i: (0, 0), memory_space=pltpu.SMEM),
        ],
        out_specs=[
            pl.BlockSpec((1, 1, BED), lambda i: (i, 0, 0)),
            pl.BlockSpec((BED, DH), lambda i: (i, 0)),
        ],
        out_shape=[
            jax.ShapeDtypeStruct((g, 1, BED), jnp.float32),
            jax.ShapeDtypeStruct((E, DH), jnp.float32),
        ],
    )(ea, wedge, bedge, wa3, m3)


def _post_body(x, aga, agb, aw, k2, cvec, woutx, wouta, bout, g1, b1,
               wp, bp, wsr, wo, bo, g2, b2, xo):
    xb = x[...]
    awb = aw[...]
    den = awb[:, DE:DE + 1]
    a16 = awb[:, :DE]
    agg_un = (jnp.concatenate([aga[...], agb[...]], axis=1)
              + a16 @ k2[...] + den * cvec[...])
    agg = (agg_un / jnp.maximum(den, 1e-30)) * (den > 0.0)
    upd = xb @ woutx[...] + agg @ wouta[...] + bout[...]
    h_ss = _ln_rows(xb + upd, g1[...], b1[...])
    h0 = jnp.tanh(h_ss @ wp[...] + bp[...])
    h1 = jnp.tanh(xb @ wp[...] + bp[...])
    sc0 = jnp.sum(h0 * wsr[...], axis=1, keepdims=True)
    sc1 = jnp.sum(h1 * wsr[...], axis=1, keepdims=True)
    mx = jnp.maximum(sc0, sc1)
    e0 = jnp.exp(sc0 - mx)
    e1 = jnp.exp(sc1 - mx)
    out = (e0 * h_ss + (2.0 * e1) * xb) / (e0 + 2.0 * e1)
    out = out @ wo[...] + bo[...]
    xo[...] = _ln_rows(out, g2[...], b2[...])


def _post(x, aga, agb, aw, k2, cvec, woutx, wouta, bout, g1, b1,
          wp, bp, wsr, wo, bo, g2, b2):
    full = lambda r, c: pl.BlockSpec((r, c), lambda i: (0, 0))
    return pl.pallas_call(
        _post_body,
        grid=(N // BN,),
        in_specs=[
            pl.BlockSpec((BN, D), lambda i: (i, 0)),
            pl.BlockSpec((BN, DH), lambda i: (i, 0)),
            pl.BlockSpec((BN, DH), lambda i: (i, 0)),
            pl.BlockSpec((BN, DH), lambda i: (i, 0)),
            full(DE, D), full(1, D), full(D, D), full(D, D), full(1, D),
            full(1, D), full(1, D), full(D, D), full(1, D), full(1, D),
            full(D, D), full(1, D), full(1, D), full(1, D),
        ],
        out_specs=pl.BlockSpec((BN, D), lambda i: (i, 0)),
        out_shape=jax.ShapeDtypeStruct((N, D), jnp.float32),
    )(x, aga, agb, aw, k2, cvec, woutx, wouta, bout, g1, b1,
      wp, bp, wsr, wo, bo, g2, b2)


def _head_body(x, w1, b1, w2r, b2, out):
    h = jnp.maximum(x[...] @ w1[...] + b1[...], 0.0)
    out[...] = (jnp.sum(h * w2r[...], axis=1) + b2[0, 0]).reshape(1, 1, BN)


def _head(x, w1, b1, w2r, b2):
    g = N // BN
    return pl.pallas_call(
        _head_body,
        grid=(g,),
        in_specs=[
            pl.BlockSpec((BN, D), lambda i: (i, 0)),
            pl.BlockSpec((D, D), lambda i: (0, 0)),
            pl.BlockSpec((1, D), lambda i: (0, 0)),
            pl.BlockSpec((1, D), lambda i: (0, 0)),
            pl.BlockSpec((1, 1), lambda i: (0, 0), memory_space=pltpu.SMEM),
        ],
        out_specs=pl.BlockSpec((1, 1, BN), lambda i: (i, 0, 0)),
        out_shape=jax.ShapeDtypeStruct((g, 1, BN), jnp.float32),
    )(x, w1, b1, w2r, b2)


def _sc_body(edata_hbm, easrc_hbm, awsrc_hbm,
             mna_hbm, mnb_hbm,
             agga_hbm, aggb_hbm, aw_hbm,
             edataA, edataB, siA, siB, diA, diB, wA, wB, easA, easB,
             pendS, pendD, pendW, pendE,
             siF, diF, wF, eidF, rowsF, awrowF, zbuf,
             acc_sh, aw_sh,
             semEA, semEB, semXA, semXB, semG, semR, semA):
    c = lax.axis_index("c")
    s = lax.axis_index("s")
    eb = s * EPT

    zeros16f = jnp.zeros((16,), jnp.float32)
    zeros16i = jnp.zeros((16,), jnp.int32)
    iota16 = lax.iota(jnp.int32, 16)
    dump16 = jnp.full((16,), NQ, jnp.int32)

    def _zrow(r, _):
        for j in range(DH // 16):
            zbuf[r, pl.ds(j * 16, 16)] = zeros16f
        return 0

    lax.fori_loop(0, ZBR, _zrow, 0)

    for j in range(PEND // 16):
        pendS[pl.ds(j * 16, 16)] = zeros16i
        pendE[pl.ds(j * 16, 16)] = zeros16i

    bufs = ((edataA, siA, diA, wA, easA, semEA, semXA),
            (edataB, siB, diB, wB, easB, semEB, semXB))

    def ed_issue(q, blk):
        pltpu.async_copy(edata_hbm.at[pl.ds((eb + blk * BE) * EW, BE * EW)],
                         bufs[q][0], bufs[q][5])

    def ed_wait(q):
        pltpu.make_async_copy(edata_hbm.at[pl.ds(0, BE * EW)],
                              bufs[q][0], bufs[q][5]).wait()

    def scores(q, lo):
        ed, si_b, di_b, w_b = bufs[q][0], bufs[q][1], bufs[q][2], bufs[q][3]
        for j in range(BE // 16):
            base = (iota16 + j * 16) * EW
            siv = plsc.load_gather(ed, [base])
            div = plsc.load_gather(ed, [base + 1])
            eaev = plsc.bitcast(plsc.load_gather(ed, [base + 2]), jnp.float32)
            w_b[pl.ds(j * 16, 16)] = eaev
            si_b[pl.ds(j * 16, 16)] = siv
            inr = (div >= lo) & (div < lo + NQ)
            di_b[pl.ds(j * 16, 16)] = jnp.where(
                inr, div - lo, NQ + (div & (NDUMP - 1)))

    def x_issue(q):
        pltpu.async_copy(easrc_hbm.at[bufs[q][1]], bufs[q][4], bufs[q][6])

    def x_wait(q):
        pltpu.make_async_copy(easrc_hbm.at[bufs[q][1]], bufs[q][4],
                              bufs[q][6]).wait()

    def wmul(q):
        w_b, eas_b = bufs[q][3], bufs[q][4]
        for j in range(BE // 16):
            w_b[pl.ds(j * 16, 16)] = (w_b[pl.ds(j * 16, 16)]
                                      * eas_b[pl.ds(j * 16, 16)])

    def compact(q, cnt, blk):
        si_b, di_b, w_b = bufs[q][1], bufs[q][2], bufs[q][3]
        base_e = eb + blk * BE
        for j in range(BE // 16):
            siv = si_b[pl.ds(j * 16, 16)]
            div = di_b[pl.ds(j * 16, 16)]
            wv = w_b[pl.ds(j * 16, 16)]
            eidv = jnp.full((16,), base_e + j * 16, jnp.int32) + iota16
            m = div < NQ
            plsc.store_compressed(pendS.at[pl.ds(cnt, 16)], siv, mask=m)
            plsc.store_compressed(pendD.at[pl.ds(cnt, 16)], div, mask=m)
            plsc.store_compressed(pendW.at[pl.ds(cnt, 16)], wv, mask=m)
            plsc.store_compressed(pendE.at[pl.ds(cnt, 16)], eidv, mask=m)
            cnt = cnt + jnp.sum(m.astype(jnp.int32))
        return cnt

    def flush(mn_hbm, do_aw):
        pltpu.make_async_copy(rowsF, acc_sh.at[diF], semR).wait()

        @pl.when(do_aw)
        def _():
            pltpu.make_async_copy(awrowF, aw_sh.at[diF], semA).wait()

        for j in range(BE // 16):
            siF[pl.ds(j * 16, 16)] = pendS[pl.ds(j * 16, 16)]
            diF[pl.ds(j * 16, 16)] = pendD[pl.ds(j * 16, 16)]
            wF[pl.ds(j * 16, 16)] = pendW[pl.ds(j * 16, 16)]
            eidF[pl.ds(j * 16, 16)] = pendE[pl.ds(j * 16, 16)]
        pltpu.async_copy(mn_hbm.at[siF], rowsF, semG)

        @pl.when(do_aw)
        def _():
            pltpu.async_copy(awsrc_hbm.at[eidF], awrowF, semG)

        pltpu.make_async_copy(mn_hbm.at[siF], rowsF, semG).wait()

        @pl.when(do_aw)
        def _():
            pltpu.make_async_copy(awsrc_hbm.at[eidF], awrowF, semG).wait()

        def _srow(i, _):
            wvec = plsc.load_gather(wF, [jnp.full((16,), i, jnp.int32)])
            for j in range(DH // 16):
                rowsF[i, pl.ds(j * 16, 16)] = rowsF[i, pl.ds(j * 16, 16)] * wvec

            @pl.when(do_aw)
            def _():
                for j in range(DH // 16):
                    awrowF[i, pl.ds(j * 16, 16)] = (
                        awrowF[i, pl.ds(j * 16, 16)] * wvec)

            return 0

        lax.fori_loop(0, BE, _srow, 0)
        pltpu.async_copy(rowsF, acc_sh.at[diF], semR, add=True)

        @pl.when(do_aw)
        def _():
            pltpu.async_copy(awrowF, aw_sh.at[diF], semA, add=True)

    def maybe_flush(cnt, mn_hbm, do_aw):
        @pl.when(cnt >= BE)
        def _():
            flush(mn_hbm, do_aw)
            for j in range(BE // 16):
                pendS[pl.ds(j * 16, 16)] = pendS[pl.ds(BE + j * 16, 16)]
                pendD[pl.ds(j * 16, 16)] = pendD[pl.ds(BE + j * 16, 16)]
                pendW[pl.ds(j * 16, 16)] = pendW[pl.ds(BE + j * 16, 16)]
                pendE[pl.ds(j * 16, 16)] = pendE[pl.ds(BE + j * 16, 16)]

        return jnp.where(cnt >= BE, cnt - BE, cnt)

    def _run(mn_hbm, agg_hbm):
        def _pass(p, _):
            lo = p * NQ
            do_aw = (p & 1) == c

            for k in range(TR // ZBR):
                pltpu.sync_copy(zbuf, acc_sh.at[pl.ds(s * TR + k * ZBR, ZBR)])

            @pl.when(do_aw)
            def _():
                for k in range(TR // ZBR):
                    pltpu.sync_copy(zbuf,
                                    aw_sh.at[pl.ds(s * TR + k * ZBR, ZBR)])

            @pl.when(s == NSUB - 1)
            def _():
                pltpu.sync_copy(zbuf.at[pl.ds(0, NDUMP)],
                                acc_sh.at[pl.ds(NQ, NDUMP)])

                @pl.when(do_aw)
                def _():
                    pltpu.sync_copy(zbuf.at[pl.ds(0, NDUMP)],
                                    aw_sh.at[pl.ds(NQ, NDUMP)])

            plsc.subcore_barrier()

            for j in range(BE // 16):
                diF[pl.ds(j * 16, 16)] = dump16
            pltpu.async_copy(rowsF, acc_sh.at[diF], semR, add=True)

            @pl.when(do_aw)
            def _():
                pltpu.async_copy(awrowF, aw_sh.at[diF], semA, add=True)

            ed_issue(0, 0)
            ed_issue(1, 1)

            def _pair(k, cnt):
                a = 2 * k
                ed_wait(0)
                scores(0, lo)
                x_issue(0)
                ed_wait(1)
                scores(1, lo)
                x_issue(1)
                x_wait(0)
                wmul(0)
                cnt = compact(0, cnt, a)
                cnt = maybe_flush(cnt, mn_hbm, do_aw)
                ed_issue(0, a + 2)
                x_wait(1)
                wmul(1)
                cnt = compact(1, cnt, a + 1)
                cnt = maybe_flush(cnt, mn_hbm, do_aw)
                ed_issue(1, a + 3)
                return cnt

            cnt = lax.fori_loop(0, (NBLK - 1) // 2, _pair, 0)

            ed_wait(0)
            scores(0, lo)
            x_issue(0)
            x_wait(0)
            wmul(0)
            cnt = compact(0, cnt, NBLK - 1)
            cnt = maybe_flush(cnt, mn_hbm, do_aw)
            ed_wait(1)

            for j in range(BE // 16):
                idxv = iota16 + j * 16
                dv = pendD[pl.ds(j * 16, 16)]
                pendD[pl.ds(j * 16, 16)] = jnp.where(idxv < cnt, dv, dump16)
            flush(mn_hbm, do_aw)

            pltpu.make_async_copy(rowsF, acc_sh.at[diF], semR).wait()

            @pl.when(do_aw)
            def _():
                pltpu.make_async_copy(awrowF, aw_sh.at[diF], semA).wait()

            plsc.subcore_barrier()
            pltpu.sync_copy(acc_sh.at[pl.ds(s * TR, TR)],
                            agg_hbm.at[pl.ds(lo + s * TR, TR)])

            @pl.when(do_aw)
            def _():
                pltpu.sync_copy(aw_sh.at[pl.ds(s * TR, TR)],
                                aw_hbm.at[pl.ds(lo + s * TR, TR)])

            plsc.subcore_barrier()
            return 0

        lax.fori_loop(0, NPASS, _pass, 0)

    @pl.when(c == 0)
    def _():
        _run(mna_hbm, agga_hbm)

    @pl.when(c == 1)
    def _():
        _run(mnb_hbm, aggb_hbm)


def _sc_aggregate(edata, easrc, awsrc, mna, mnb):
    mesh = plsc.VectorSubcoreMesh(core_axis_name="c", subcore_axis_name="s",
                                  num_cores=2, num_subcores=NSUB)
    k = pl.kernel(
        _sc_body,
        out_type=[
            jax.ShapeDtypeStruct((NP, DH), jnp.float32),
            jax.ShapeDtypeStruct((NP, DH), jnp.float32),
            jax.ShapeDtypeStruct((NP, DH), jnp.float32),
        ],
        mesh=mesh,
        compiler_params=pltpu.CompilerParams(needs_layout_passes=False),
        scratch_types=(
            [pltpu.VMEM((BE * EW,), jnp.int32) for _ in range(2)]
            + [pltpu.VMEM((BE,), jnp.int32) for _ in range(4)]
            + [pltpu.VMEM((BE,), jnp.float32) for _ in range(4)]
            + [pltpu.VMEM((PEND,), jnp.int32) for _ in range(2)]
            + [pltpu.VMEM((PEND,), jnp.float32)]
            + [pltpu.VMEM((PEND,), jnp.int32)]
            + [pltpu.VMEM((BE,), jnp.int32) for _ in range(2)]
            + [pltpu.VMEM((BE,), jnp.float32)]
            + [pltpu.VMEM((BE,), jnp.int32)]
            + [pltpu.VMEM((BE, DH), jnp.float32) for _ in range(2)]
            + [pltpu.VMEM((ZBR, DH), jnp.float32)]
            + [pltpu.VMEM_SHARED((NQ + NDUMP, DH), jnp.float32)] * 2
            + [pltpu.SemaphoreType.DMA] * 7
        ),
    )
    return k(edata, easrc, awsrc, mna, mnb)


def kernel(stock_feat, edge_index, edge_attr, params):
    p = params
    si = edge_index[0]
    di = edge_index[1]

    r = lambda v: v.reshape(1, -1)

    x = _enc(stock_feat, p['enc_W1'], r(p['enc_b1']), p['enc_W2'], r(p['enc_b2']))

    for l in range(L):
        wsrc, bsrc = p[f'l{l}_Wsrc'], p[f'l{l}_bsrc']
        wedge, bedge = p[f'l{l}_Wedge'], p[f'l{l}_bedge']
        wattn = p[f'l{l}_Wattn']
        wmsg, bmsg = p[f'l{l}_Wmsg'], p[f'l{l}_bmsg']
        wa1, wa3 = wattn[:D, 0], wattn[2 * D:, 0]

        wsm = wsrc @ wmsg
        bsm = bsrc @ wmsg
        k2 = wedge @ wmsg
        cvec = bedge @ wmsg + bmsg
        m1 = jnp.sum(jnp.abs(wa1)).reshape(1, 1)
        m3 = jnp.sum(jnp.abs(wa3)).reshape(1, 1)
        wout = p[f'l{l}_Wout']

        mna, mnb, easrc3 = _pre_node(x, wsrc, r(bsrc), wsm, r(bsm), r(wa1), m1)
        eae3, awsrc = _pre_edge(edge_attr, wedge, r(bedge), r(wa3), m3)

        ebase = jnp.stack(
            [si, di, jax.lax.bitcast_convert_type(eae3.reshape(E), jnp.int32),
             jnp.zeros((E,), jnp.int32)], axis=1)
        edata = jnp.concatenate(
            [ebase, jnp.zeros((BE, EW), jnp.int32)],
            axis=0).reshape((E + BE) * EW)

        agga_p, aggb_p, aw_p = _sc_aggregate(edata, easrc3.reshape(N), awsrc,
                                             mna, mnb)
        agga, aggb, aw = agga_p[:N], aggb_p[:N], aw_p[:N]

        x = _post(
            x, agga, aggb, aw, k2, r(cvec), wout[:D], wout[D:], r(p[f'l{l}_bout']),
            r(p[f'l{l}_g']), r(p[f'l{l}_b']),
            p['mp_Wp'], r(p['mp_bp']), r(p['mp_Ws'][:, 0]),
            p['mp_Wo'], r(p['mp_bo']), r(p['mp_g']), r(p['mp_b']))

    logits3 = _head(x, p['head_W1'], r(p['head_b1']), r(p['head_W2'][:, 0]),
                    p['head_b2'].reshape(1, 1))
    return logits3.reshape(N)


# speedup vs baseline: 3.2562x; 1.0333x over previous
import functools

import jax
import jax.numpy as jnp
from jax import lax
from jax.experimental import pallas as pl
from jax.experimental.pallas import tpu as pltpu
from jax.experimental.pallas import tpu_sc as plsc

N, E, D_IN, D, DE, L = 10000, 160000, 256, 256, 16, 2

BN = 2000
BED = 3200
NSUB = 16
EPT = E // NSUB
BE = 80
NBLK = EPT // BE
DH = D // 2
NQ = 5120
NPASS = 2
NP = NQ * NPASS
NDUMP = 8
TR = NQ // NSUB
ZBR = 16
EW = 4
FB = 128
PEND = 224


def _ln_rows(h, g, b):
    mu = jnp.mean(h, axis=1, keepdims=True)
    var = jnp.mean((h - mu) ** 2, axis=1, keepdims=True)
    return (h - mu) * jax.lax.rsqrt(var + 1e-5) * g + b


def _enc_body(sf, w1, b1, w2, b2, out):
    h = jnp.maximum(sf[...] @ w1[...] + b1[...], 0.0)
    out[...] = h @ w2[...] + b2[...]


def _enc(sf, w1, b1, w2, b2):
    return pl.pallas_call(
        _enc_body,
        grid=(N // BN,),
        in_specs=[
            pl.BlockSpec((BN, D_IN), lambda i: (i, 0)),
            pl.BlockSpec((D_IN, D), lambda i: (0, 0)),
            pl.BlockSpec((1, D), lambda i: (0, 0)),
            pl.BlockSpec((D, D), lambda i: (0, 0)),
            pl.BlockSpec((1, D), lambda i: (0, 0)),
        ],
        out_specs=pl.BlockSpec((BN, D), lambda i: (i, 0)),
        out_shape=jax.ShapeDtypeStruct((N, D), jnp.float32),
    )(sf, w1, b1, w2, b2)


def _pre_node_body(x, wsrc, bsrc, wsm, bsm, wa1, m1, mna, mnb, easrc):
    xb = x[...]
    p = xb @ wsrc[...] + bsrc[...]
    mn = xb @ wsm[...] + bsm[...]
    mna[...] = mn[:, :DH]
    mnb[...] = mn[:, DH:]
    a = jnp.sum(jnp.tanh(p) * wa1[...], axis=1) - m1[0, 0]
    easrc[...] = jnp.exp(a).reshape(1, 1, BN)


def _pre_node(x, wsrc, bsrc, wsm, bsm, wa1, m1):
    g = N // BN
    return pl.pallas_call(
        _pre_node_body,
        grid=(g,),
        in_specs=[
            pl.BlockSpec((BN, D), lambda i: (i, 0)),
            pl.BlockSpec((D, D), lambda i: (0, 0)),
            pl.BlockSpec((1, D), lambda i: (0, 0)),
            pl.BlockSpec((D, D), lambda i: (0, 0)),
            pl.BlockSpec((1, D), lambda i: (0, 0)),
            pl.BlockSpec((1, D), lambda i: (0, 0)),
            pl.BlockSpec((1, 1), lambda i: (0, 0), memory_space=pltpu.SMEM),
        ],
        out_specs=[
            pl.BlockSpec((BN, DH), lambda i: (i, 0)),
            pl.BlockSpec((BN, DH), lambda i: (i, 0)),
            pl.BlockSpec((1, 1, BN), lambda i: (i, 0, 0)),
        ],
        out_shape=[
            jax.ShapeDtypeStruct((N, DH), jnp.float32),
            jax.ShapeDtypeStruct((N, DH), jnp.float32),
            jax.ShapeDtypeStruct((g, 1, BN), jnp.float32),
        ],
    )(x, wsrc, bsrc, wsm, bsm, wa1, m1)


def _pre_edge_body(ea, wedge, bedge, wa3, m3, eae, awsrc):
    eab = ea[...]
    s = jnp.tanh(eab @ wedge[...] + bedge[...])
    a = jnp.sum(s * wa3[...], axis=1) - m3[0, 0]
    eae[...] = jnp.exp(a).reshape(1, 1, BED)
    awsrc[...] = jnp.concatenate(
        [eab, jnp.ones((BED, 1), jnp.float32),
         jnp.zeros((BED, DH - DE - 1), jnp.float32)], axis=1)


def _pre_edge(ea, wedge, bedge, wa3, m3):
    g = E // BED
    return pl.pallas_call(
        _pre_edge_body,
        grid=(g,),
        in_specs=[
            pl.BlockSpec((BED, DE), lambda i: (i, 0)),
            pl.BlockSpec((DE, D), lambda i: (0, 0)),
            pl.BlockSpec((1, D), lambda i: (0, 0)),
            pl.BlockSpec((1, D), lambda i: (0, 0)),
            pl.BlockSpec((1, 1), lambda i: (0, 0), memory_space=pltpu.SMEM),
        ],
        out_specs=[
            pl.BlockSpec((1, 1, BED), lambda i: (i, 0, 0)),
            pl.BlockSpec((BED, DH), lambda i: (i, 0)),
        ],
        out_shape=[
            jax.ShapeDtypeStruct((g, 1, BED), jnp.float32),
            jax.ShapeDtypeStruct((E, DH), jnp.float32),
        ],
    )(ea, wedge, bedge, wa3, m3)


def _post_body(x, aga, agb, aw, k2, cvec, woutx, wouta, bout, g1, b1,
               wp, bp, wsr, wo, bo, g2, b2, xo):
    xb = x[...]
    awb = aw[...]
    den = awb[:, DE:DE + 1]
    a16 = awb[:, :DE]
    agg_un = (jnp.concatenate([aga[...], agb[...]], axis=1)
              + a16 @ k2[...] + den * cvec[...])
    agg = (agg_un / jnp.maximum(den, 1e-30)) * (den > 0.0)
    upd = xb @ woutx[...] + agg @ wouta[...] + bout[...]
    h_ss = _ln_rows(xb + upd, g1[...], b1[...])
    h0 = jnp.tanh(h_ss @ wp[...] + bp[...])
    h1 = jnp.tanh(xb @ wp[...] + bp[...])
    sc0 = jnp.sum(h0 * wsr[...], axis=1, keepdims=True)
    sc1 = jnp.sum(h1 * wsr[...], axis=1, keepdims=True)
    mx = jnp.maximum(sc0, sc1)
    e0 = jnp.exp(sc0 - mx)
    e1 = jnp.exp(sc1 - mx)
    out = (e0 * h_ss + (2.0 * e1) * xb) / (e0 + 2.0 * e1)
    out = out @ wo[...] + bo[...]
    xo[...] = _ln_rows(out, g2[...], b2[...])


def _post(x, aga, agb, aw, k2, cvec, woutx, wouta, bout, g1, b1,
          wp, bp, wsr, wo, bo, g2, b2):
    full = lambda r, c: pl.BlockSpec((r, c), lambda i: (0, 0))
    return pl.pallas_call(
        _post_body,
        grid=(N // BN,),
        in_specs=[
            pl.BlockSpec((BN, D), lambda i: (i, 0)),
            pl.BlockSpec((BN, DH), lambda i: (i, 0)),
            pl.BlockSpec((BN, DH), lambda i: (i, 0)),
            pl.BlockSpec((BN, DH), lambda i: (i, 0)),
            full(DE, D), full(1, D), full(D, D), full(D, D), full(1, D),
            full(1, D), full(1, D), full(D, D), full(1, D), full(1, D),
            full(D, D), full(1, D), full(1, D), full(1, D),
        ],
        out_specs=pl.BlockSpec((BN, D), lambda i: (i, 0)),
        out_shape=jax.ShapeDtypeStruct((N, D), jnp.float32),
    )(x, aga, agb, aw, k2, cvec, woutx, wouta, bout, g1, b1,
      wp, bp, wsr, wo, bo, g2, b2)


def _head_body(x, w1, b1, w2r, b2, out):
    h = jnp.maximum(x[...] @ w1[...] + b1[...], 0.0)
    out[...] = (jnp.sum(h * w2r[...], axis=1) + b2[0, 0]).reshape(1, 1, BN)


def _head(x, w1, b1, w2r, b2):
    g = N // BN
    return pl.pallas_call(
        _head_body,
        grid=(g,),
        in_specs=[
            pl.BlockSpec((BN, D), lambda i: (i, 0)),
            pl.BlockSpec((D, D), lambda i: (0, 0)),
            pl.BlockSpec((1, D), lambda i: (0, 0)),
            pl.BlockSpec((1, D), lambda i: (0, 0)),
            pl.BlockSpec((1, 1), lambda i: (0, 0), memory_space=pltpu.SMEM),
        ],
        out_specs=pl.BlockSpec((1, 1, BN), lambda i: (i, 0, 0)),
        out_shape=jax.ShapeDtypeStruct((g, 1, BN), jnp.float32),
    )(x, w1, b1, w2r, b2)


def _sc_body(edata_hbm, easrc_hbm, awsrc_hbm,
             mna_hbm, mnb_hbm,
             agga_hbm, aggb_hbm, aw_hbm,
             edataA, edataB, siA, siB, diA, diB, wA, wB, easA, easB,
             pendS, pendD, pendW, pendE,
             siF, diF, wF, eidF, rowsF, awrowF, zbuf,
             acc_sh, aw_sh,
             semEA, semEB, semXA, semXB, semG, semR, semA):
    c = lax.axis_index("c")
    s = lax.axis_index("s")
    eb = s * EPT

    zeros16f = jnp.zeros((16,), jnp.float32)
    zeros16i = jnp.zeros((16,), jnp.int32)
    iota16 = lax.iota(jnp.int32, 16)
    dump16 = jnp.full((16,), NQ, jnp.int32)

    def _zrow(r, _):
        for j in range(DH // 16):
            zbuf[r, pl.ds(j * 16, 16)] = zeros16f
        return 0

    lax.fori_loop(0, ZBR, _zrow, 0)

    for j in range(PEND // 16):
        pendS[pl.ds(j * 16, 16)] = zeros16i
        pendE[pl.ds(j * 16, 16)] = zeros16i

    bufs = ((edataA, siA, diA, wA, easA, semEA, semXA),
            (edataB, siB, diB, wB, easB, semEB, semXB))

    def ed_issue(q, blk):
        pltpu.async_copy(edata_hbm.at[pl.ds((eb + blk * BE) * EW, BE * EW)],
                         bufs[q][0], bufs[q][5])

    def ed_wait(q):
        pltpu.make_async_copy(edata_hbm.at[pl.ds(0, BE * EW)],
                              bufs[q][0], bufs[q][5]).wait()

    def scores(q, lo):
        ed, si_b, di_b, w_b = bufs[q][0], bufs[q][1], bufs[q][2], bufs[q][3]
        for j in range(BE // 16):
            base = (iota16 + j * 16) * EW
            siv = plsc.load_gather(ed, [base])
            div = plsc.load_gather(ed, [base + 1])
            eaev = plsc.bitcast(plsc.load_gather(ed, [base + 2]), jnp.float32)
            w_b[pl.ds(j * 16, 16)] = eaev
            si_b[pl.ds(j * 16, 16)] = siv
            inr = (div >= lo) & (div < lo + NQ)
            di_b[pl.ds(j * 16, 16)] = jnp.where(
                inr, div - lo, NQ + (div & (NDUMP - 1)))

    def x_issue(q):
        pltpu.async_copy(easrc_hbm.at[bufs[q][1]], bufs[q][4], bufs[q][6])

    def x_wait(q):
        pltpu.make_async_copy(easrc_hbm.at[bufs[q][1]], bufs[q][4],
                              bufs[q][6]).wait()

    def wmul(q):
        w_b, eas_b = bufs[q][3], bufs[q][4]
        for j in range(BE // 16):
            w_b[pl.ds(j * 16, 16)] = (w_b[pl.ds(j * 16, 16)]
                                      * eas_b[pl.ds(j * 16, 16)])

    def compact(q, cnt, blk):
        si_b, di_b, w_b = bufs[q][1], bufs[q][2], bufs[q][3]
        base_e = eb + blk * BE
        for j in range(BE // 16):
            siv = si_b[pl.ds(j * 16, 16)]
            div = di_b[pl.ds(j * 16, 16)]
            wv = w_b[pl.ds(j * 16, 16)]
            eidv = jnp.full((16,), base_e + j * 16, jnp.int32) + iota16
            m = div < NQ
            plsc.store_compressed(pendS.at[pl.ds(cnt, 16)], siv, mask=m)
            plsc.store_compressed(pendD.at[pl.ds(cnt, 16)], div, mask=m)
            plsc.store_compressed(pendW.at[pl.ds(cnt, 16)], wv, mask=m)
            plsc.store_compressed(pendE.at[pl.ds(cnt, 16)], eidv, mask=m)
            cnt = cnt + jnp.sum(m.astype(jnp.int32))
        return cnt

    def flush(mn_hbm, do_aw):
        pltpu.make_async_copy(rowsF, acc_sh.at[diF], semR).wait()

        @pl.when(do_aw)
        def _():
            pltpu.make_async_copy(awrowF, aw_sh.at[diF], semA).wait()

        for j in range(FB // 16):
            siF[pl.ds(j * 16, 16)] = pendS[pl.ds(j * 16, 16)]
            diF[pl.ds(j * 16, 16)] = pendD[pl.ds(j * 16, 16)]
            wF[pl.ds(j * 16, 16)] = pendW[pl.ds(j * 16, 16)]
            eidF[pl.ds(j * 16, 16)] = pendE[pl.ds(j * 16, 16)]
        pltpu.async_copy(mn_hbm.at[siF], rowsF, semG)

        @pl.when(do_aw)
        def _():
            pltpu.async_copy(awsrc_hbm.at[eidF], awrowF, semG)

        pltpu.make_async_copy(mn_hbm.at[siF], rowsF, semG).wait()

        @pl.when(do_aw)
        def _():
            pltpu.make_async_copy(awsrc_hbm.at[eidF], awrowF, semG).wait()

        def _srow(i, _):
            wvec = plsc.load_gather(wF, [jnp.full((16,), i, jnp.int32)])
            for j in range(DH // 16):
                rowsF[i, pl.ds(j * 16, 16)] = rowsF[i, pl.ds(j * 16, 16)] * wvec

            @pl.when(do_aw)
            def _():
                for j in range(DH // 16):
                    awrowF[i, pl.ds(j * 16, 16)] = (
                        awrowF[i, pl.ds(j * 16, 16)] * wvec)

            return 0

        lax.fori_loop(0, FB, _srow, 0)
        pltpu.async_copy(rowsF, acc_sh.at[diF], semR, add=True)

        @pl.when(do_aw)
        def _():
            pltpu.async_copy(awrowF, aw_sh.at[diF], semA, add=True)

    def maybe_flush(cnt, mn_hbm, do_aw):
        @pl.when(cnt >= FB)
        def _():
            flush(mn_hbm, do_aw)
            for j in range(BE // 16):
                pendS[pl.ds(j * 16, 16)] = pendS[pl.ds(FB + j * 16, 16)]
                pendD[pl.ds(j * 16, 16)] = pendD[pl.ds(FB + j * 16, 16)]
                pendW[pl.ds(j * 16, 16)] = pendW[pl.ds(FB + j * 16, 16)]
                pendE[pl.ds(j * 16, 16)] = pendE[pl.ds(FB + j * 16, 16)]

        return jnp.where(cnt >= FB, cnt - FB, cnt)

    def _run(mn_hbm, agg_hbm):
        def _pass(p, _):
            lo = p * NQ
            do_aw = (p & 1) == c

            for k in range(TR // ZBR):
                pltpu.sync_copy(zbuf, acc_sh.at[pl.ds(s * TR + k * ZBR, ZBR)])

            @pl.when(do_aw)
            def _():
                for k in range(TR // ZBR):
                    pltpu.sync_copy(zbuf,
                                    aw_sh.at[pl.ds(s * TR + k * ZBR, ZBR)])

            @pl.when(s == NSUB - 1)
            def _():
                pltpu.sync_copy(zbuf.at[pl.ds(0, NDUMP)],
                                acc_sh.at[pl.ds(NQ, NDUMP)])

                @pl.when(do_aw)
                def _():
                    pltpu.sync_copy(zbuf.at[pl.ds(0, NDUMP)],
                                    aw_sh.at[pl.ds(NQ, NDUMP)])

            plsc.subcore_barrier()

            for j in range(FB // 16):
                diF[pl.ds(j * 16, 16)] = dump16
            pltpu.async_copy(rowsF, acc_sh.at[diF], semR, add=True)

            @pl.when(do_aw)
            def _():
                pltpu.async_copy(awrowF, aw_sh.at[diF], semA, add=True)

            ed_issue(0, 0)
            ed_issue(1, 1)

            def _pair(k, cnt):
                a = 2 * k
                ed_wait(0)
                scores(0, lo)
                x_issue(0)
                ed_wait(1)
                scores(1, lo)
                x_issue(1)
                x_wait(0)
                wmul(0)
                cnt = compact(0, cnt, a)
                cnt = maybe_flush(cnt, mn_hbm, do_aw)
                ed_issue(0, a + 2)
                x_wait(1)
                wmul(1)
                cnt = compact(1, cnt, a + 1)
                cnt = maybe_flush(cnt, mn_hbm, do_aw)
                ed_issue(1, a + 3)
                return cnt

            cnt = lax.fori_loop(0, (NBLK - 1) // 2, _pair, 0)

            ed_wait(0)
            scores(0, lo)
            x_issue(0)
            x_wait(0)
            wmul(0)
            cnt = compact(0, cnt, NBLK - 1)
            cnt = maybe_flush(cnt, mn_hbm, do_aw)
            ed_wait(1)

            for j in range(FB // 16):
                idxv = iota16 + j * 16
                dv = pendD[pl.ds(j * 16, 16)]
                pendD[pl.ds(j * 16, 16)] = jnp.where(idxv < cnt, dv, dump16)
            flush(mn_hbm, do_aw)

            pltpu.make_async_copy(rowsF, acc_sh.at[diF], semR).wait()

            @pl.when(do_aw)
            def _():
                pltpu.make_async_copy(awrowF, aw_sh.at[diF], semA).wait()

            plsc.subcore_barrier()
            pltpu.sync_copy(acc_sh.at[pl.ds(s * TR, TR)],
                            agg_hbm.at[pl.ds(lo + s * TR, TR)])

            @pl.when(do_aw)
            def _():
                pltpu.sync_copy(aw_sh.at[pl.ds(s * TR, TR)],
                                aw_hbm.at[pl.ds(lo + s * TR, TR)])

            plsc.subcore_barrier()
            return 0

        lax.fori_loop(0, NPASS, _pass, 0)

    @pl.when(c == 0)
    def _():
        _run(mna_hbm, agga_hbm)

    @pl.when(c == 1)
    def _():
        _run(mnb_hbm, aggb_hbm)


def _sc_aggregate(edata, easrc, awsrc, mna, mnb):
    mesh = plsc.VectorSubcoreMesh(core_axis_name="c", subcore_axis_name="s",
                                  num_cores=2, num_subcores=NSUB)
    k = pl.kernel(
        _sc_body,
        out_type=[
            jax.ShapeDtypeStruct((NP, DH), jnp.float32),
            jax.ShapeDtypeStruct((NP, DH), jnp.float32),
            jax.ShapeDtypeStruct((NP, DH), jnp.float32),
        ],
        mesh=mesh,
        compiler_params=pltpu.CompilerParams(needs_layout_passes=False),
        scratch_types=(
            [pltpu.VMEM((BE * EW,), jnp.int32) for _ in range(2)]
            + [pltpu.VMEM((BE,), jnp.int32) for _ in range(4)]
            + [pltpu.VMEM((BE,), jnp.float32) for _ in range(4)]
            + [pltpu.VMEM((PEND,), jnp.int32) for _ in range(2)]
            + [pltpu.VMEM((PEND,), jnp.float32)]
            + [pltpu.VMEM((PEND,), jnp.int32)]
            + [pltpu.VMEM((FB,), jnp.int32) for _ in range(2)]
            + [pltpu.VMEM((FB,), jnp.float32)]
            + [pltpu.VMEM((FB,), jnp.int32)]
            + [pltpu.VMEM((FB, DH), jnp.float32) for _ in range(2)]
            + [pltpu.VMEM((ZBR, DH), jnp.float32)]
            + [pltpu.VMEM_SHARED((NQ + NDUMP, DH), jnp.float32)] * 2
            + [pltpu.SemaphoreType.DMA] * 7
        ),
    )
    return k(edata, easrc, awsrc, mna, mnb)


def kernel(stock_feat, edge_index, edge_attr, params):
    p = params
    si = edge_index[0]
    di = edge_index[1]

    r = lambda v: v.reshape(1, -1)

    x = _enc(stock_feat, p['enc_W1'], r(p['enc_b1']), p['enc_W2'], r(p['enc_b2']))

    for l in range(L):
        wsrc, bsrc = p[f'l{l}_Wsrc'], p[f'l{l}_bsrc']
        wedge, bedge = p[f'l{l}_Wedge'], p[f'l{l}_bedge']
        wattn = p[f'l{l}_Wattn']
        wmsg, bmsg = p[f'l{l}_Wmsg'], p[f'l{l}_bmsg']
        wa1, wa3 = wattn[:D, 0], wattn[2 * D:, 0]

        wsm = wsrc @ wmsg
        bsm = bsrc @ wmsg
        k2 = wedge @ wmsg
        cvec = bedge @ wmsg + bmsg
        m1 = jnp.sum(jnp.abs(wa1)).reshape(1, 1)
        m3 = jnp.sum(jnp.abs(wa3)).reshape(1, 1)
        wout = p[f'l{l}_Wout']

        mna, mnb, easrc3 = _pre_node(x, wsrc, r(bsrc), wsm, r(bsm), r(wa1), m1)
        eae3, awsrc = _pre_edge(edge_attr, wedge, r(bedge), r(wa3), m3)

        ebase = jnp.stack(
            [si, di, jax.lax.bitcast_convert_type(eae3.reshape(E), jnp.int32),
             jnp.zeros((E,), jnp.int32)], axis=1)
        edata = jnp.concatenate(
            [ebase, jnp.zeros((BE, EW), jnp.int32)],
            axis=0).reshape((E + BE) * EW)

        agga_p, aggb_p, aw_p = _sc_aggregate(edata, easrc3.reshape(N), awsrc,
                                             mna, mnb)
        agga, aggb, aw = agga_p[:N], aggb_p[:N], aw_p[:N]

        x = _post(
            x, agga, aggb, aw, k2, r(cvec), wout[:D], wout[D:], r(p[f'l{l}_bout']),
            r(p[f'l{l}_g']), r(p[f'l{l}_b']),
            p['mp_Wp'], r(p['mp_bp']), r(p['mp_Ws'][:, 0]),
            p['mp_Wo'], r(p['mp_bo']), r(p['mp_g']), r(p['mp_b']))

    logits3 = _head(x, p['head_W1'], r(p['head_b1']), r(p['head_W2'][:, 0]),
                    p['head_b2'].reshape(1, 1))
    return logits3.reshape(N)


# speedup vs baseline: 3.2566x; 1.0001x over previous
import jax
import jax.numpy as jnp
from jax import lax
from jax.experimental import pallas as pl
from jax.experimental.pallas import tpu as pltpu
from jax.experimental.pallas import tpu_sc as plsc

N, E, D_IN, D, DE, L = 10000, 160000, 256, 256, 16, 2

BN = 2000
BED = 3200
NSUB = 16
EPT = E // NSUB
BE = 80
NBLK = EPT // BE
DH = D // 2
NQ = 5120
NPASS = 2
NP = NQ * NPASS
NDUMP = 8
TR = NQ // NSUB
ZBR = 16
EW = 4
FB = 128
PEND = 224


def _ln_rows(h, g, b):
    mu = jnp.mean(h, axis=1, keepdims=True)
    var = jnp.mean((h - mu) ** 2, axis=1, keepdims=True)
    return (h - mu) * jax.lax.rsqrt(var + 1e-5) * g + b


def _enc_body(sf, w1, b1, w2, b2, out):
    h = jnp.maximum(sf[...] @ w1[...] + b1[...], 0.0)
    out[...] = h @ w2[...] + b2[...]


def _enc(sf, w1, b1, w2, b2):
    return pl.pallas_call(
        _enc_body,
        grid=(N // BN,),
        in_specs=[
            pl.BlockSpec((BN, D_IN), lambda i: (i, 0)),
            pl.BlockSpec((D_IN, D), lambda i: (0, 0)),
            pl.BlockSpec((1, D), lambda i: (0, 0)),
            pl.BlockSpec((D, D), lambda i: (0, 0)),
            pl.BlockSpec((1, D), lambda i: (0, 0)),
        ],
        out_specs=pl.BlockSpec((BN, D), lambda i: (i, 0)),
        out_shape=jax.ShapeDtypeStruct((N, D), jnp.float32),
    )(sf, w1, b1, w2, b2)


def _pre_node_body(x, wsrc, bsrc, wsm, bsm, wa1, m1, mna, mnb, easrc):
    xb = x[...]
    p = xb @ wsrc[...] + bsrc[...]
    mn = xb @ wsm[...] + bsm[...]
    mna[...] = mn[:, :DH]
    mnb[...] = mn[:, DH:]
    a = jnp.sum(jnp.tanh(p) * wa1[...], axis=1) - m1[0, 0]
    easrc[...] = jnp.exp(a).reshape(1, 1, BN)


def _pre_node(x, wsrc, bsrc, wsm, bsm, wa1, m1):
    g = N // BN
    return pl.pallas_call(
        _pre_node_body,
        grid=(g,),
        in_specs=[
            pl.BlockSpec((BN, D), lambda i: (i, 0)),
            pl.BlockSpec((D, D), lambda i: (0, 0)),
            pl.BlockSpec((1, D), lambda i: (0, 0)),
            pl.BlockSpec((D, D), lambda i: (0, 0)),
            pl.BlockSpec((1, D), lambda i: (0, 0)),
            pl.BlockSpec((1, D), lambda i: (0, 0)),
            pl.BlockSpec((1, 1), lambda i: (0, 0), memory_space=pltpu.SMEM),
        ],
        out_specs=[
            pl.BlockSpec((BN, DH), lambda i: (i, 0)),
            pl.BlockSpec((BN, DH), lambda i: (i, 0)),
            pl.BlockSpec((1, 1, BN), lambda i: (i, 0, 0)),
        ],
        out_shape=[
            jax.ShapeDtypeStruct((N, DH), jnp.float32),
            jax.ShapeDtypeStruct((N, DH), jnp.float32),
            jax.ShapeDtypeStruct((g, 1, BN), jnp.float32),
        ],
    )(x, wsrc, bsrc, wsm, bsm, wa1, m1)


def _pre_edge_body(ea, wedge, bedge, wa3, m3, eae, awsrc):
    eab = ea[...]
    s = jnp.tanh(eab @ wedge[...] + bedge[...])
    a = jnp.sum(s * wa3[...], axis=1) - m3[0, 0]
    eae[...] = jnp.exp(a).reshape(1, 1, BED)
    awsrc[...] = jnp.concatenate(
        [eab, jnp.ones((BED, 1), jnp.float32),
         jnp.zeros((BED, DH - DE - 1), jnp.float32)], axis=1)


def _pre_edge(ea, wedge, bedge, wa3, m3):
    g = E // BED
    return pl.pallas_call(
        _pre_edge_body,
        grid=(g,),
        in_specs=[
            pl.BlockSpec((BED, DE), lambda i: (i, 0)),
            pl.BlockSpec((DE, D), lambda i: (0, 0)),
            pl.BlockSpec((1, D), lambda i: (0, 0)),
            pl.BlockSpec((1, D), lambda i: (0, 0)),
            pl.BlockSpec((1, 1), lambda i: (0, 0), memory_space=pltpu.SMEM),
        ],
        out_specs=[
            pl.BlockSpec((1, 1, BED), lambda i: (i, 0, 0)),
            pl.BlockSpec((BED, DH), lambda i: (i, 0)),
        ],
        out_shape=[
            jax.ShapeDtypeStruct((g, 1, BED), jnp.float32),
            jax.ShapeDtypeStruct((E, DH), jnp.float32),
        ],
    )(ea, wedge, bedge, wa3, m3)


def _post_body(x, aga, agb, aw, k2, cvec, woutx, wouta, bout, g1, b1,
               wp, bp, wsr, wo, bo, g2, b2, xo):
    xb = x[...]
    awb = aw[...]
    den = awb[:, DE:DE + 1]
    a16 = awb[:, :DE]
    agg_un = (jnp.concatenate([aga[...], agb[...]], axis=1)
              + a16 @ k2[...] + den * cvec[...])
    agg = (agg_un / jnp.maximum(den, 1e-30)) * (den > 0.0)
    upd = xb @ woutx[...] + agg @ wouta[...] + bout[...]
    h_ss = _ln_rows(xb + upd, g1[...], b1[...])
    h0 = jnp.tanh(h_ss @ wp[...] + bp[...])
    h1 = jnp.tanh(xb @ wp[...] + bp[...])
    sc0 = jnp.sum(h0 * wsr[...], axis=1, keepdims=True)
    sc1 = jnp.sum(h1 * wsr[...], axis=1, keepdims=True)
    mx = jnp.maximum(sc0, sc1)
    e0 = jnp.exp(sc0 - mx)
    e1 = jnp.exp(sc1 - mx)
    out = (e0 * h_ss + (2.0 * e1) * xb) / (e0 + 2.0 * e1)
    out = out @ wo[...] + bo[...]
    xo[...] = _ln_rows(out, g2[...], b2[...])


def _post(x, aga, agb, aw, k2, cvec, woutx, wouta, bout, g1, b1,
          wp, bp, wsr, wo, bo, g2, b2):
    full = lambda r, c: pl.BlockSpec((r, c), lambda i: (0, 0))
    return pl.pallas_call(
        _post_body,
        grid=(N // BN,),
        in_specs=[
            pl.BlockSpec((BN, D), lambda i: (i, 0)),
            pl.BlockSpec((BN, DH), lambda i: (i, 0)),
            pl.BlockSpec((BN, DH), lambda i: (i, 0)),
            pl.BlockSpec((BN, DH), lambda i: (i, 0)),
            full(DE, D), full(1, D), full(D, D), full(D, D), full(1, D),
            full(1, D), full(1, D), full(D, D), full(1, D), full(1, D),
            full(D, D), full(1, D), full(1, D), full(1, D),
        ],
        out_specs=pl.BlockSpec((BN, D), lambda i: (i, 0)),
        out_shape=jax.ShapeDtypeStruct((N, D), jnp.float32),
    )(x, aga, agb, aw, k2, cvec, woutx, wouta, bout, g1, b1,
      wp, bp, wsr, wo, bo, g2, b2)


def _head_body(x, w1, b1, w2r, b2, out):
    h = jnp.maximum(x[...] @ w1[...] + b1[...], 0.0)
    out[...] = (jnp.sum(h * w2r[...], axis=1) + b2[0, 0]).reshape(1, 1, BN)


def _head(x, w1, b1, w2r, b2):
    g = N // BN
    return pl.pallas_call(
        _head_body,
        grid=(g,),
        in_specs=[
            pl.BlockSpec((BN, D), lambda i: (i, 0)),
            pl.BlockSpec((D, D), lambda i: (0, 0)),
            pl.BlockSpec((1, D), lambda i: (0, 0)),
            pl.BlockSpec((1, D), lambda i: (0, 0)),
            pl.BlockSpec((1, 1), lambda i: (0, 0), memory_space=pltpu.SMEM),
        ],
        out_specs=pl.BlockSpec((1, 1, BN), lambda i: (i, 0, 0)),
        out_shape=jax.ShapeDtypeStruct((g, 1, BN), jnp.float32),
    )(x, w1, b1, w2r, b2)


def _sc_body(edata_hbm, easrc_hbm, awsrc_hbm,
             mna_hbm, mnb_hbm,
             agga_hbm, aggb_hbm, aw_hbm,
             edataA, edataB, siA, siB, diA, diB, wA, wB, easA, easB,
             pendS, pendD, pendW, pendE,
             siF, diF, wF, eidF, rowsF, awrowF, zbuf,
             acc_sh, aw_sh,
             semEA, semEB, semXA, semXB, semG, semR, semA):
    c = lax.axis_index("c")
    s = lax.axis_index("s")
    eb = s * EPT

    zeros16f = jnp.zeros((16,), jnp.float32)
    zeros16i = jnp.zeros((16,), jnp.int32)
    iota16 = lax.iota(jnp.int32, 16)
    dump16 = jnp.full((16,), NQ, jnp.int32)

    def _zrow(r, _):
        for j in range(DH // 16):
            zbuf[r, pl.ds(j * 16, 16)] = zeros16f
        return 0

    lax.fori_loop(0, ZBR, _zrow, 0)

    for j in range(PEND // 16):
        pendS[pl.ds(j * 16, 16)] = zeros16i
        pendE[pl.ds(j * 16, 16)] = zeros16i

    bufs = ((edataA, siA, diA, wA, easA, semEA, semXA),
            (edataB, siB, diB, wB, easB, semEB, semXB))

    def ed_issue(q, blk):
        pltpu.async_copy(edata_hbm.at[pl.ds((eb + blk * BE) * EW, BE * EW)],
                         bufs[q][0], bufs[q][5])

    def ed_wait(q):
        pltpu.make_async_copy(edata_hbm.at[pl.ds(0, BE * EW)],
                              bufs[q][0], bufs[q][5]).wait()

    def scores(q, lo):
        ed, si_b, di_b, w_b = bufs[q][0], bufs[q][1], bufs[q][2], bufs[q][3]
        for j in range(BE // 16):
            base = (iota16 + j * 16) * EW
            siv = plsc.load_gather(ed, [base])
            div = plsc.load_gather(ed, [base + 1])
            eaev = plsc.bitcast(plsc.load_gather(ed, [base + 2]), jnp.float32)
            w_b[pl.ds(j * 16, 16)] = eaev
            si_b[pl.ds(j * 16, 16)] = siv
            inr = (div >= lo) & (div < lo + NQ)
            di_b[pl.ds(j * 16, 16)] = jnp.where(
                inr, div - lo, NQ + (div & (NDUMP - 1)))

    def x_issue(q):
        pltpu.async_copy(easrc_hbm.at[bufs[q][1]], bufs[q][4], bufs[q][6])

    def x_wait(q):
        pltpu.make_async_copy(easrc_hbm.at[bufs[q][1]], bufs[q][4],
                              bufs[q][6]).wait()

    def wmul(q):
        w_b, eas_b = bufs[q][3], bufs[q][4]
        for j in range(BE // 16):
            w_b[pl.ds(j * 16, 16)] = (w_b[pl.ds(j * 16, 16)]
                                      * eas_b[pl.ds(j * 16, 16)])

    def compact(q, cnt, blk):
        si_b, di_b, w_b = bufs[q][1], bufs[q][2], bufs[q][3]
        base_e = eb + blk * BE
        for j in range(BE // 16):
            siv = si_b[pl.ds(j * 16, 16)]
            div = di_b[pl.ds(j * 16, 16)]
            wv = w_b[pl.ds(j * 16, 16)]
            eidv = jnp.full((16,), base_e + j * 16, jnp.int32) + iota16
            m = div < NQ
            plsc.store_compressed(pendS.at[pl.ds(cnt, 16)], siv, mask=m)
            plsc.store_compressed(pendD.at[pl.ds(cnt, 16)], div, mask=m)
            plsc.store_compressed(pendW.at[pl.ds(cnt, 16)], wv, mask=m)
            plsc.store_compressed(pendE.at[pl.ds(cnt, 16)], eidv, mask=m)
            cnt = cnt + jnp.sum(m.astype(jnp.int32))
        return cnt

    def flush(mn_hbm, do_aw):
        pltpu.make_async_copy(rowsF, acc_sh.at[diF], semR).wait()

        @pl.when(do_aw)
        def _():
            pltpu.make_async_copy(awrowF, aw_sh.at[diF], semA).wait()

        for j in range(FB // 16):
            siF[pl.ds(j * 16, 16)] = pendS[pl.ds(j * 16, 16)]
            diF[pl.ds(j * 16, 16)] = pendD[pl.ds(j * 16, 16)]
            wF[pl.ds(j * 16, 16)] = pendW[pl.ds(j * 16, 16)]
            eidF[pl.ds(j * 16, 16)] = pendE[pl.ds(j * 16, 16)]
        pltpu.async_copy(mn_hbm.at[siF], rowsF, semG)

        @pl.when(do_aw)
        def _():
            pltpu.async_copy(awsrc_hbm.at[eidF], awrowF, semG)

        pltpu.make_async_copy(mn_hbm.at[siF], rowsF, semG).wait()

        @pl.when(do_aw)
        def _():
            pltpu.make_async_copy(awsrc_hbm.at[eidF], awrowF, semG).wait()

        def _srow(i, _):
            wvec = plsc.load_gather(wF, [jnp.full((16,), i, jnp.int32)])
            for j in range(DH // 16):
                rowsF[i, pl.ds(j * 16, 16)] = rowsF[i, pl.ds(j * 16, 16)] * wvec

            @pl.when(do_aw)
            def _():
                for j in range(DH // 16):
                    awrowF[i, pl.ds(j * 16, 16)] = (
                        awrowF[i, pl.ds(j * 16, 16)] * wvec)

            return 0

        lax.fori_loop(0, FB, _srow, 0)
        pltpu.async_copy(rowsF, acc_sh.at[diF], semR, add=True)

        @pl.when(do_aw)
        def _():
            pltpu.async_copy(awrowF, aw_sh.at[diF], semA, add=True)

    def maybe_flush(cnt, mn_hbm, do_aw):
        @pl.when(cnt >= FB)
        def _():
            flush(mn_hbm, do_aw)
            for j in range(BE // 16):
                pendS[pl.ds(j * 16, 16)] = pendS[pl.ds(FB + j * 16, 16)]
                pendD[pl.ds(j * 16, 16)] = pendD[pl.ds(FB + j * 16, 16)]
                pendW[pl.ds(j * 16, 16)] = pendW[pl.ds(FB + j * 16, 16)]
                pendE[pl.ds(j * 16, 16)] = pendE[pl.ds(FB + j * 16, 16)]

        return jnp.where(cnt >= FB, cnt - FB, cnt)

    def _run(mn_hbm, agg_hbm):
        def _pass(p, _):
            lo = p * NQ
            do_aw = (p & 1) == c

            for k in range(TR // ZBR):
                pltpu.sync_copy(zbuf, acc_sh.at[pl.ds(s * TR + k * ZBR, ZBR)])

            @pl.when(do_aw)
            def _():
                for k in range(TR // ZBR):
                    pltpu.sync_copy(zbuf,
                                    aw_sh.at[pl.ds(s * TR + k * ZBR, ZBR)])

            @pl.when(s == NSUB - 1)
            def _():
                pltpu.sync_copy(zbuf.at[pl.ds(0, NDUMP)],
                                acc_sh.at[pl.ds(NQ, NDUMP)])

                @pl.when(do_aw)
                def _():
                    pltpu.sync_copy(zbuf.at[pl.ds(0, NDUMP)],
                                    aw_sh.at[pl.ds(NQ, NDUMP)])

            plsc.subcore_barrier()

            for j in range(FB // 16):
                diF[pl.ds(j * 16, 16)] = dump16
            pltpu.async_copy(rowsF, acc_sh.at[diF], semR, add=True)

            @pl.when(do_aw)
            def _():
                pltpu.async_copy(awrowF, aw_sh.at[diF], semA, add=True)

            ed_issue(0, 0)
            ed_issue(1, 1)

            def _pair(k, cnt):
                a = 2 * k
                ed_wait(0)
                scores(0, lo)
                x_issue(0)
                ed_wait(1)
                scores(1, lo)
                x_issue(1)
                x_wait(0)
                wmul(0)
                cnt = compact(0, cnt, a)
                cnt = maybe_flush(cnt, mn_hbm, do_aw)
                ed_issue(0, a + 2)
                x_wait(1)
                wmul(1)
                cnt = compact(1, cnt, a + 1)
                cnt = maybe_flush(cnt, mn_hbm, do_aw)
                ed_issue(1, a + 3)
                return cnt

            cnt = lax.fori_loop(0, (NBLK - 1) // 2, _pair, 0)

            ed_wait(0)
            scores(0, lo)
            x_issue(0)
            x_wait(0)
            wmul(0)
            cnt = compact(0, cnt, NBLK - 1)
            cnt = maybe_flush(cnt, mn_hbm, do_aw)
            ed_wait(1)

            for j in range(FB // 16):
                idxv = iota16 + j * 16
                dv = pendD[pl.ds(j * 16, 16)]
                pendD[pl.ds(j * 16, 16)] = jnp.where(idxv < cnt, dv, dump16)
            flush(mn_hbm, do_aw)

            pltpu.make_async_copy(rowsF, acc_sh.at[diF], semR).wait()

            @pl.when(do_aw)
            def _():
                pltpu.make_async_copy(awrowF, aw_sh.at[diF], semA).wait()

            plsc.subcore_barrier()
            pltpu.sync_copy(acc_sh.at[pl.ds(s * TR, TR)],
                            agg_hbm.at[pl.ds(lo + s * TR, TR)])

            @pl.when(do_aw)
            def _():
                pltpu.sync_copy(aw_sh.at[pl.ds(s * TR, TR)],
                                aw_hbm.at[pl.ds(lo + s * TR, TR)])

            plsc.subcore_barrier()
            return 0

        lax.fori_loop(0, NPASS, _pass, 0)

    @pl.when(c == 0)
    def _():
        _run(mna_hbm, agga_hbm)

    @pl.when(c == 1)
    def _():
        _run(mnb_hbm, aggb_hbm)


def _sc_aggregate(edata, easrc, awsrc, mna, mnb):
    mesh = plsc.VectorSubcoreMesh(core_axis_name="c", subcore_axis_name="s",
                                  num_cores=2, num_subcores=NSUB)
    k = pl.kernel(
        _sc_body,
        out_type=[
            jax.ShapeDtypeStruct((NP, DH), jnp.float32),
            jax.ShapeDtypeStruct((NP, DH), jnp.float32),
            jax.ShapeDtypeStruct((NP, DH), jnp.float32),
        ],
        mesh=mesh,
        compiler_params=pltpu.CompilerParams(needs_layout_passes=False),
        scratch_types=(
            [pltpu.VMEM((BE * EW,), jnp.int32) for _ in range(2)]
            + [pltpu.VMEM((BE,), jnp.int32) for _ in range(4)]
            + [pltpu.VMEM((BE,), jnp.float32) for _ in range(4)]
            + [pltpu.VMEM((PEND,), jnp.int32) for _ in range(2)]
            + [pltpu.VMEM((PEND,), jnp.float32)]
            + [pltpu.VMEM((PEND,), jnp.int32)]
            + [pltpu.VMEM((FB,), jnp.int32) for _ in range(2)]
            + [pltpu.VMEM((FB,), jnp.float32)]
            + [pltpu.VMEM((FB,), jnp.int32)]
            + [pltpu.VMEM((FB, DH), jnp.float32) for _ in range(2)]
            + [pltpu.VMEM((ZBR, DH), jnp.float32)]
            + [pltpu.VMEM_SHARED((NQ + NDUMP, DH), jnp.float32)] * 2
            + [pltpu.SemaphoreType.DMA] * 7
        ),
    )
    return k(edata, easrc, awsrc, mna, mnb)


def kernel(stock_feat, edge_index, edge_attr, params):
    p = params
    si = edge_index[0]
    di = edge_index[1]

    r = lambda v: v.reshape(1, -1)

    x = _enc(stock_feat, p['enc_W1'], r(p['enc_b1']), p['enc_W2'], r(p['enc_b2']))

    for l in range(L):
        wsrc, bsrc = p[f'l{l}_Wsrc'], p[f'l{l}_bsrc']
        wedge, bedge = p[f'l{l}_Wedge'], p[f'l{l}_bedge']
        wattn = p[f'l{l}_Wattn']
        wmsg, bmsg = p[f'l{l}_Wmsg'], p[f'l{l}_bmsg']
        wa1, wa3 = wattn[:D, 0], wattn[2 * D:, 0]

        wsm = wsrc @ wmsg
        bsm = bsrc @ wmsg
        k2 = wedge @ wmsg
        cvec = bedge @ wmsg + bmsg
        m1 = jnp.sum(jnp.abs(wa1)).reshape(1, 1)
        m3 = jnp.sum(jnp.abs(wa3)).reshape(1, 1)
        wout = p[f'l{l}_Wout']

        mna, mnb, easrc3 = _pre_node(x, wsrc, r(bsrc), wsm, r(bsm), r(wa1), m1)
        eae3, awsrc = _pre_edge(edge_attr, wedge, r(bedge), r(wa3), m3)

        ebase = jnp.stack(
            [si, di, jax.lax.bitcast_convert_type(eae3.reshape(E), jnp.int32),
             jnp.zeros((E,), jnp.int32)], axis=1)
        edata = jnp.concatenate(
            [ebase, jnp.zeros((BE, EW), jnp.int32)],
            axis=0).reshape((E + BE) * EW)

        agga_p, aggb_p, aw_p = _sc_aggregate(edata, easrc3.reshape(N), awsrc,
                                             mna, mnb)
        agga, aggb, aw = agga_p[:N], aggb_p[:N], aw_p[:N]

        x = _post(
            x, agga, aggb, aw, k2, r(cvec), wout[:D], wout[D:], r(p[f'l{l}_bout']),
            r(p[f'l{l}_g']), r(p[f'l{l}_b']),
            p['mp_Wp'], r(p['mp_bp']), r(p['mp_Ws'][:, 0]),
            p['mp_Wo'], r(p['mp_bo']), r(p['mp_g']), r(p['mp_b']))

    logits3 = _head(x, p['head_W1'], r(p['head_b1']), r(p['head_W2'][:, 0]),
                    p['head_b2'].reshape(1, 1))
    return logits3.reshape(N)
